# baseline (speedup 1.0000x reference)
import functools

import jax
import jax.numpy as jnp
from jax import lax
from jax.experimental import pallas as pl
from jax.experimental.pallas import tpu as pltpu
from jax.experimental.pallas import tpu_sc as plsc

N = 50000
K = 16
D = 64
NK = N * K
EPS = 1e-5

BN2 = 1000
PN = 400
PR2 = PN * K // 2
RB8 = 4000
CHUNK = 128
NW = 32
NCHUNKS = NK // CHUNK


def _lrelu(x):
    return jnp.where(x >= 0, x, 0.2 * x)


def _dup(v):
    return jnp.concatenate([v, v])


def _bd2(w):
    a, b = w.shape
    z = jnp.zeros((2 * a, 2 * b), w.dtype)
    return z.at[:a, :b].set(w).at[a:, b:].set(w)


def _sc_gather(fpre, xyzp, idx):
    mesh = plsc.VectorSubcoreMesh(core_axis_name="c", subcore_axis_name="s")

    @functools.partial(
        pl.kernel,
        mesh=mesh,
        compiler_params=pltpu.CompilerParams(use_tc_tiling_on_sc=False),
        out_type=[
            jax.ShapeDtypeStruct((NK, D), jnp.bfloat16),
            jax.ShapeDtypeStruct((NK, 16), jnp.float32),
        ],
        scratch_types=[
            pltpu.VMEM((CHUNK,), jnp.int32),
            pltpu.VMEM((CHUNK, D), jnp.bfloat16),
            pltpu.VMEM((CHUNK, 16), jnp.float32),
            pltpu.SemaphoreType.DMA,
            pltpu.SemaphoreType.DMA,
        ],
    )
    def k(fpre_hbm, xyzp_hbm, idx_hbm, g_hbm, y_hbm, idx_v, rows_v, yrows_v,
          sem1, sem2):
        wid = lax.axis_index("s") * 2 + lax.axis_index("c")

        def body(j, carry):
            c = j * NW + wid

            @pl.when(c < NCHUNKS)
            def _():
                base = c * CHUNK
                pltpu.sync_copy(idx_hbm.at[pl.ds(base, CHUNK)], idx_v)
                cp1 = pltpu.async_copy(fpre_hbm.at[idx_v], rows_v, sem1)
                cp2 = pltpu.async_copy(xyzp_hbm.at[idx_v], yrows_v, sem2)
                cp1.wait()
                cp2.wait()
                pltpu.sync_copy(rows_v, g_hbm.at[pl.ds(base, CHUNK)])
                pltpu.sync_copy(yrows_v, y_hbm.at[pl.ds(base, CHUNK)])

            return carry

        lax.fori_loop(0, (NCHUNKS + NW - 1) // NW, body, 0)

    return k(fpre, xyzp, idx)


def _acc_stats(i, stats_ref, acc, s1, s2):
    @pl.when(i == 0)
    def _():
        acc[...] = jnp.zeros_like(acc)

    acc[0:1, :] += s1
    acc[1:2, :] += s2

    @pl.when(i == pl.num_programs(0) - 1)
    def _():
        stats_ref[...] = acc[...]


def _pass_a(x2, w1big):
    def body(x_ref, w_ref, fpre_ref, fpreb_ref, stats_ref, acc):
        i = pl.program_id(0)
        t = jnp.dot(x_ref[...], w_ref[...], preferred_element_type=jnp.float32)
        fpre_ref[...] = t
        fpreb_ref[...] = t.astype(jnp.bfloat16)
        _acc_stats(i, stats_ref, acc,
                   jnp.sum(t, axis=0, keepdims=True),
                   jnp.sum(t * t, axis=0, keepdims=True))

    return pl.pallas_call(
        body,
        grid=(N // 2 // BN2,),
        in_specs=[
            pl.BlockSpec((BN2, 2 * D), lambda i: (i, 0)),
            pl.BlockSpec((2 * D, 2 * D), lambda i: (0, 0)),
        ],
        out_specs=[
            pl.BlockSpec((BN2, 2 * D), lambda i: (i, 0)),
            pl.BlockSpec((BN2, 2 * D), lambda i: (i, 0)),
            pl.BlockSpec((2, 2 * D), lambda i: (0, 0)),
        ],
        out_shape=[
            jax.ShapeDtypeStruct((N // 2, 2 * D), jnp.float32),
            jax.ShapeDtypeStruct((N // 2, 2 * D), jnp.bfloat16),
            jax.ShapeDtypeStruct((2, 2 * D), jnp.float32),
        ],
        scratch_shapes=[pltpu.VMEM((2, 2 * D), jnp.float32)],
    )(x2, w1big)


def _pass_b0(y8, wp1big8):
    def body(y_ref, w_ref, stats_ref, acc):
        i = pl.program_id(0)
        t = jnp.dot(y_ref[...], w_ref[...], preferred_element_type=jnp.float32)
        _acc_stats(i, stats_ref, acc,
                   jnp.sum(t, axis=0, keepdims=True),
                   jnp.sum(t * t, axis=0, keepdims=True))

    return pl.pallas_call(
        body,
        grid=(NK // 8 // RB8,),
        in_specs=[
            pl.BlockSpec((RB8, 128), lambda i: (i, 0)),
            pl.BlockSpec((128, 32), lambda i: (0, 0)),
        ],
        out_specs=pl.BlockSpec((2, 32), lambda i: (0, 0)),
        out_shape=jax.ShapeDtypeStruct((2, 32), jnp.float32),
        scratch_shapes=[pltpu.VMEM((2, 32), jnp.float32)],
    )(y8, wp1big8)


def _rqk_block(g_ref, y_ref, fp_ref, wq_ref, wk_ref, wp1_ref, wp2_ref,
               cv_ref, c8_ref):
    a1 = cv_ref[0:1, :]
    b1 = cv_ref[1:2, :]
    bq = cv_ref[2:3, :]
    bk = cv_ref[3:4, :]
    bp2 = cv_ref[4:5, :]
    a2 = c8_ref[0:1, :]
    b2 = c8_ref[1:2, :]

    f2 = jnp.maximum(fp_ref[...] * a1 + b1, 0.0)
    q2 = jnp.dot(f2, wq_ref[...], preferred_element_type=jnp.float32) + bq
    qb = jnp.broadcast_to(q2[None], (K, PN // 2, 2 * D)).reshape(PR2, 2 * D)
    fg = jnp.maximum(g_ref[...].astype(jnp.float32) * a1 + b1, 0.0)
    fg = fg.reshape(PR2, 2 * D)
    kg = jnp.dot(fg, wk_ref[...], preferred_element_type=jnp.float32) + bk
    y = y_ref[...].reshape(PR2, 32)
    t = jnp.dot(y, wp1_ref[...], preferred_element_type=jnp.float32)
    t = jnp.maximum(t * a2 + b2, 0.0)
    pr = jnp.dot(t, wp2_ref[...], preferred_element_type=jnp.float32) + bp2
    r = kg - qb + pr
    return r, fg, pr


_GATHER_SPECS = [
    pl.BlockSpec((K, PN // 2, 2 * D), lambda i: (0, i, 0)),
    pl.BlockSpec((K, PN // 2, 32), lambda i: (0, i, 0)),
    pl.BlockSpec((PN // 2, 2 * D), lambda i: (i, 0)),
    pl.BlockSpec((2 * D, 2 * D), lambda i: (0, 0)),
    pl.BlockSpec((2 * D, 2 * D), lambda i: (0, 0)),
    pl.BlockSpec((32, 8), lambda i: (0, 0)),
    pl.BlockSpec((8, 2 * D), lambda i: (0, 0)),
    pl.BlockSpec((8, 2 * D), lambda i: (0, 0)),
    pl.BlockSpec((2, 8), lambda i: (0, 0)),
]


def _pass_b(g2, y2, fpre, wqt, wkbig, wp1big, wp2big, cv, c8, wvbig):
    def body(g_ref, y_ref, fpre_ref, wq_ref, wk_ref, wp1_ref, wp2_ref, cv_ref,
             c8_ref, wv_ref, rb_ref, svb_ref, stats_ref, acc):
        i = pl.program_id(0)
        r, fg, pr = _rqk_block(g_ref, y_ref, fpre_ref, wq_ref, wk_ref, wp1_ref,
                               wp2_ref, cv_ref, c8_ref)
        bv = cv_ref[5:6, :]
        vg = jnp.dot(fg, wv_ref[...], preferred_element_type=jnp.float32) + bv
        rb_ref[...] = r.reshape(K, PN // 2, 2 * D).astype(jnp.bfloat16)
        svb_ref[...] = (vg + pr).reshape(K, PN // 2, 2 * D).astype(jnp.bfloat16)
        _acc_stats(i, stats_ref, acc,
                   jnp.sum(r, axis=0, keepdims=True),
                   jnp.sum(r * r, axis=0, keepdims=True))

    return pl.pallas_call(
        body,
        grid=(N // PN,),
        in_specs=_GATHER_SPECS + [
            pl.BlockSpec((2 * D, 2 * D), lambda i: (0, 0)),
        ],
        out_specs=[
            pl.BlockSpec((K, PN // 2, 2 * D), lambda i: (0, i, 0)),
            pl.BlockSpec((K, PN // 2, 2 * D), lambda i: (0, i, 0)),
            pl.BlockSpec((2, 2 * D), lambda i: (0, 0)),
        ],
        out_shape=[
            jax.ShapeDtypeStruct((K, N // 2, 2 * D), jnp.bfloat16),
            jax.ShapeDtypeStruct((K, N // 2, 2 * D), jnp.bfloat16),
            jax.ShapeDtypeStruct((2, 2 * D), jnp.float32),
        ],
        scratch_shapes=[pltpu.VMEM((2, 2 * D), jnp.float32)],
    )(g2, y2, fpre, wqt, wkbig, wp1big, wp2big, cv, c8, wvbig)


def _pass_c(rb, ww1big, c3):
    def body(rb_ref, ww1_ref, c3_ref, stats_ref, acc):
        i = pl.program_id(0)
        r = rb_ref[...].reshape(PR2, 2 * D).astype(jnp.float32)
        u = _lrelu(r * c3_ref[0:1, :] + c3_ref[1:2, :])
        w1 = jnp.dot(u, ww1_ref[...], preferred_element_type=jnp.float32)
        _acc_stats(i, stats_ref, acc,
                   jnp.sum(w1, axis=0, keepdims=True),
                   jnp.sum(w1 * w1, axis=0, keepdims=True))

    return pl.pallas_call(
        body,
        grid=(N // PN,),
        in_specs=[
            pl.BlockSpec((K, PN // 2, 2 * D), lambda i: (0, i, 0)),
            pl.BlockSpec((2 * D, 16), lambda i: (0, 0)),
            pl.BlockSpec((2, 2 * D), lambda i: (0, 0)),
        ],
        out_specs=pl.BlockSpec((2, 16), lambda i: (0, 0)),
        out_shape=jax.ShapeDtypeStruct((2, 16), jnp.float32),
        scratch_shapes=[pltpu.VMEM((2, 16), jnp.float32)],
    )(rb, ww1big, c3)


def _pass_d(rb, svb, ww1big, c3, ww2big, c16, sel16):
    def body(rb_ref, svb_ref, ww1_ref, c3_ref, ww2_ref, c16_ref, sel16_ref,
             x_ref, stats_ref, acc):
        i = pl.program_id(0)
        r = rb_ref[...].reshape(PR2, 2 * D).astype(jnp.float32)
        u = _lrelu(r * c3_ref[0:1, :] + c3_ref[1:2, :])
        w1 = jnp.dot(u, ww1_ref[...], preferred_element_type=jnp.float32)
        u4 = jnp.maximum(w1 * c16_ref[0:1, :] + c16_ref[1:2, :], 0.0)
        w2 = jnp.dot(u4, ww2_ref[...], preferred_element_type=jnp.float32)
        w2 = w2 + c16_ref[2:3, :]
        e = jnp.exp(w2)
        efull = jnp.dot(e, sel16_ref[...],
                        preferred_element_type=jnp.float32,
                        precision=jax.lax.Precision.HIGHEST)
        sv = svb_ref[...].reshape(PR2, 2 * D).astype(jnp.float32)
        p3 = (sv * efull).reshape(K, PN // 2, 2 * D)
        numer = jnp.sum(p3, axis=0)
        zf = jnp.sum(efull.reshape(K, PN // 2, 2 * D), axis=0)
        x = numer / zf
        x_ref[...] = x
        _acc_stats(i, stats_ref, acc,
                   jnp.sum(x, axis=0, keepdims=True),
                   jnp.sum(x * x, axis=0, keepdims=True))

    return pl.pallas_call(
        body,
        grid=(N // PN,),
        in_specs=[
            pl.BlockSpec((K, PN // 2, 2 * D), lambda i: (0, i, 0)),
            pl.BlockSpec((K, PN // 2, 2 * D), lambda i: (0, i, 0)),
            pl.BlockSpec((2 * D, 16), lambda i: (0, 0)),
            pl.BlockSpec((2, 2 * D), lambda i: (0, 0)),
            pl.BlockSpec((16, 16), lambda i: (0, 0)),
            pl.BlockSpec((4, 16), lambda i: (0, 0)),
            pl.BlockSpec((16, 2 * D), lambda i: (0, 0)),
        ],
        out_specs=[
            pl.BlockSpec((PN // 2, 2 * D), lambda i: (i, 0)),
            pl.BlockSpec((2, 2 * D), lambda i: (0, 0)),
        ],
        out_shape=[
            jax.ShapeDtypeStruct((N // 2, 2 * D), jnp.float32),
            jax.ShapeDtypeStruct((2, 2 * D), jnp.float32),
        ],
        scratch_shapes=[pltpu.VMEM((2, 2 * D), jnp.float32)],
    )(rb, svb, ww1big, c3, ww2big, c16, sel16)


def _pass_e2(xagg2, wc2big, c5):
    def body(x_ref, w_ref, c_ref, x2_ref, stats_ref, acc):
        i = pl.program_id(0)
        u = _lrelu(x_ref[...] * c_ref[0:1, :] + c_ref[1:2, :])
        x2 = jnp.dot(u, w_ref[...], preferred_element_type=jnp.float32)
        x2_ref[...] = x2
        _acc_stats(i, stats_ref, acc,
                   jnp.sum(x2, axis=0, keepdims=True),
                   jnp.sum(x2 * x2, axis=0, keepdims=True))

    return pl.pallas_call(
        body,
        grid=(N // 2 // BN2,),
        in_specs=[
            pl.BlockSpec((BN2, 2 * D), lambda i: (i, 0)),
            pl.BlockSpec((2 * D, 2 * D), lambda i: (0, 0)),
            pl.BlockSpec((2, 2 * D), lambda i: (0, 0)),
        ],
        out_specs=[
            pl.BlockSpec((BN2, 2 * D), lambda i: (i, 0)),
            pl.BlockSpec((2, 2 * D), lambda i: (0, 0)),
        ],
        out_shape=[
            jax.ShapeDtypeStruct((N // 2, 2 * D), jnp.float32),
            jax.ShapeDtypeStruct((2, 2 * D), jnp.float32),
        ],
        scratch_shapes=[pltpu.VMEM((2, 2 * D), jnp.float32)],
    )(xagg2, wc2big, c5)


def _pass_e3(fpre2, x22, ce):
    def body(fpre_ref, x2_ref, c_ref, out_ref):
        f = jnp.maximum(fpre_ref[...] * c_ref[0:1, :] + c_ref[1:2, :], 0.0)
        xb = x2_ref[...] * c_ref[2:3, :] + c_ref[3:4, :]
        out_ref[...] = _lrelu(f + xb)

    return pl.pallas_call(
        body,
        grid=(N // 2 // BN2,),
        in_specs=[
            pl.BlockSpec((BN2, 2 * D), lambda i: (i, 0)),
            pl.BlockSpec((BN2, 2 * D), lambda i: (i, 0)),
            pl.BlockSpec((4, 2 * D), lambda i: (0, 0)),
        ],
        out_specs=pl.BlockSpec((BN2, 2 * D), lambda i: (i, 0)),
        out_shape=jax.ShapeDtypeStruct((N // 2, 2 * D), jnp.float32),
    )(fpre2, x22, ce)


def _bn_affine(g, b, s1, s2, m):
    mean = s1 / m
    var = s2 / m - mean * mean
    a = g / jnp.sqrt(var + EPS)
    return a, b - mean * a


def kernel(feature, xyz, params, neigh_idx):
    p = params
    x2 = feature[0, :, :, 0].T.reshape(N // 2, 2 * D)
    xyzp = jnp.pad(xyz[0], ((0, 0), (0, 13)))
    idx = neigh_idx[0].T.reshape(-1).astype(jnp.int32)

    fpre2, fpreb2, st1r = _pass_a(x2, _bd2(p['W1'].T))
    st1 = st1r[:, :D] + st1r[:, D:]
    a1, b1 = _bn_affine(p['g1'], p['b1'], st1[0], st1[1], N)

    g_flat, y_flat = _sc_gather(fpreb2.reshape(N, D), xyzp, idx)
    g2 = g_flat.reshape(K, N // 2, 2 * D)
    y2 = y_flat.reshape(K, N // 2, 32)
    y8 = y_flat.reshape(NK // 8, 128)

    wp1t16 = jnp.zeros((16, 4), jnp.float32).at[:3, :3].set(p['Wp1'].T)
    wp1big8 = jax.scipy.linalg.block_diag(*([wp1t16] * 8))
    st2r = _pass_b0(y8, wp1big8)
    st2 = st2r.reshape(2, 8, 4).sum(axis=1)
    g2p = jnp.pad(p['gp1'], (0, 1))
    b2p = jnp.pad(p['bp1'], (0, 1))
    a2, b2 = _bn_affine(g2p, b2p, st2[0], st2[1], NK)

    cv = jnp.stack([_dup(a1), _dup(b1),
                    _dup(p['bq']),
                    _dup(p['bk']), _dup(p['bp2']), _dup(p['bv']),
                    jnp.zeros(2 * D, jnp.float32),
                    jnp.zeros(2 * D, jnp.float32)])
    c8 = jnp.stack([_dup(a2), _dup(b2)])
    wqt = _bd2(p['Wq'].T)
    wkbig = _bd2(p['Wk'].T)
    wp1big = _bd2(wp1t16)
    wp2big = _bd2(jnp.pad(p['Wp2'].T, ((0, 1), (0, 0))))

    rb, svb, st3r = _pass_b(g2, y2, fpre2, wqt, wkbig, wp1big, wp2big, cv, c8,
                            _bd2(p['Wv'].T))
    st3 = st3r[:, :D] + st3r[:, D:]
    a3, b3 = _bn_affine(p['gw0'], p['bw0'], st3[0], st3[1], NK)
    c3 = jnp.stack([_dup(a3), _dup(b3)])

    ww1big = _bd2(p['Ww1'].T)
    st4r = _pass_c(rb, ww1big, c3)
    st4 = st4r[:, :8] + st4r[:, 8:]
    a4, b4 = _bn_affine(p['gw1'], p['bw1'], st4[0], st4[1], NK)

    c16 = jnp.stack([_dup(a4), _dup(b4), _dup(p['bw2']),
                     jnp.zeros(16, jnp.float32)])
    lanes = jnp.arange(2 * D)
    sel16 = (jnp.arange(16)[:, None]
             == jnp.where(lanes < D, lanes % 8, 8 + lanes % 8)[None, :]
             ).astype(jnp.float32)
    xagg2, st5r = _pass_d(rb, svb, ww1big, c3, _bd2(p['Ww2'].T), c16, sel16)
    st5 = st5r[:, :D] + st5r[:, D:]
    a5, b5 = _bn_affine(p['g_bn'], p['b_bn'], st5[0], st5[1], N)

    c5 = jnp.stack([_dup(a5), _dup(b5)])
    x22, st6r = _pass_e2(xagg2, _bd2(p['Wc2'].T), c5)
    st6 = st6r[:, :D] + st6r[:, D:]
    a6, b6 = _bn_affine(p['gc2'], p['bc2'], st6[0], st6[1], N)

    ce = jnp.stack([_dup(a1), _dup(b1), _dup(a6), _dup(b6)])
    out = _pass_e3(fpre2, x22, ce).reshape(N, D)
    return out.T[None, :, :, None]


# speedup vs baseline: 1.4100x; 1.4100x over previous
import functools

import jax
import jax.numpy as jnp
from jax import lax
from jax.experimental import pallas as pl
from jax.experimental.pallas import tpu as pltpu
from jax.experimental.pallas import tpu_sc as plsc

N = 50000
K = 16
D = 64
NK = N * K
EPS = 1e-5

BN2 = 1000
PN = 400
PR2 = PN * K // 2
RB8 = 4000
CHUNK = 128
NW = 32
NCHUNKS = NK // CHUNK


def _lrelu(x):
    return jnp.where(x >= 0, x, 0.2 * x)


def _dup(v):
    return jnp.concatenate([v, v])


def _bd2(w):
    a, b = w.shape
    z = jnp.zeros((2 * a, 2 * b), w.dtype)
    return z.at[:a, :b].set(w).at[a:, b:].set(w)


def _sc_gather(fpre, xyzp, idx):
    mesh = plsc.VectorSubcoreMesh(core_axis_name="c", subcore_axis_name="s")
    nj = (NCHUNKS + NW - 1) // NW

    @functools.partial(
        pl.kernel,
        mesh=mesh,
        compiler_params=pltpu.CompilerParams(use_tc_tiling_on_sc=False),
        out_type=[
            jax.ShapeDtypeStruct((NK, D), jnp.float32),
            jax.ShapeDtypeStruct((NK, 16), jnp.float32),
        ],
        scratch_types=[
            pltpu.VMEM((CHUNK,), jnp.int32),
            pltpu.VMEM((CHUNK,), jnp.int32),
            pltpu.VMEM((CHUNK, D), jnp.float32),
            pltpu.VMEM((CHUNK, D), jnp.float32),
            pltpu.VMEM((CHUNK, 16), jnp.float32),
            pltpu.VMEM((CHUNK, 16), jnp.float32),
            pltpu.SemaphoreType.DMA,
            pltpu.SemaphoreType.DMA,
            pltpu.SemaphoreType.DMA,
            pltpu.SemaphoreType.DMA,
        ],
    )
    def k(fpre_hbm, xyzp_hbm, idx_hbm, g_hbm, y_hbm,
          idx0, idx1, rows0, rows1, yrows0, yrows1,
          gsem0, gsem1, ysem0, ysem1):
        wid = lax.axis_index("s") * 2 + lax.axis_index("c")
        idxs = (idx0, idx1)
        rows = (rows0, rows1)
        yrows = (yrows0, yrows1)
        gsems = (gsem0, gsem1)
        ysems = (ysem0, ysem1)

        def issue(j, b):
            c = j * NW + wid

            @pl.when(c < NCHUNKS)
            def _():
                base = c * CHUNK
                pltpu.sync_copy(idx_hbm.at[pl.ds(base, CHUNK)], idxs[b])
                pltpu.async_copy(fpre_hbm.at[idxs[b]], rows[b], gsems[b])
                pltpu.async_copy(xyzp_hbm.at[idxs[b]], yrows[b], ysems[b])

        def drain(j, b):
            c = j * NW + wid

            @pl.when(c < NCHUNKS)
            def _():
                base = c * CHUNK
                pltpu.make_async_copy(fpre_hbm.at[idxs[b]], rows[b],
                                      gsems[b]).wait()
                pltpu.make_async_copy(xyzp_hbm.at[idxs[b]], yrows[b],
                                      ysems[b]).wait()
                pltpu.sync_copy(rows[b], g_hbm.at[pl.ds(base, CHUNK)])
                pltpu.sync_copy(yrows[b], y_hbm.at[pl.ds(base, CHUNK)])

        issue(0, 0)

        def body(j2, carry):
            j0 = j2 * 2
            issue(j0 + 1, 1)
            drain(j0, 0)
            issue(j0 + 2, 0)
            drain(j0 + 1, 1)
            return carry

        lax.fori_loop(0, nj // 2, body, 0)

    return k(fpre, xyzp, idx)


def _acc_stats(i, stats_ref, acc, s1, s2):
    @pl.when(i == 0)
    def _():
        acc[...] = jnp.zeros_like(acc)

    acc[0:1, :] += s1
    acc[1:2, :] += s2

    @pl.when(i == pl.num_programs(0) - 1)
    def _():
        stats_ref[...] = acc[...]


def _pass_a(x2, w1big):
    def body(x_ref, w_ref, fpre_ref, stats_ref, acc):
        i = pl.program_id(0)
        t = jnp.dot(x_ref[...], w_ref[...], preferred_element_type=jnp.float32)
        fpre_ref[...] = t
        _acc_stats(i, stats_ref, acc,
                   jnp.sum(t, axis=0, keepdims=True),
                   jnp.sum(t * t, axis=0, keepdims=True))

    return pl.pallas_call(
        body,
        grid=(N // 2 // BN2,),
        in_specs=[
            pl.BlockSpec((BN2, 2 * D), lambda i: (i, 0)),
            pl.BlockSpec((2 * D, 2 * D), lambda i: (0, 0)),
        ],
        out_specs=[
            pl.BlockSpec((BN2, 2 * D), lambda i: (i, 0)),
            pl.BlockSpec((2, 2 * D), lambda i: (0, 0)),
        ],
        out_shape=[
            jax.ShapeDtypeStruct((N // 2, 2 * D), jnp.float32),
            jax.ShapeDtypeStruct((2, 2 * D), jnp.float32),
        ],
        scratch_shapes=[pltpu.VMEM((2, 2 * D), jnp.float32)],
    )(x2, w1big)


def _pass_b0(y8, wp1big8):
    def body(y_ref, w_ref, stats_ref, acc):
        i = pl.program_id(0)
        t = jnp.dot(y_ref[...], w_ref[...], preferred_element_type=jnp.float32)
        _acc_stats(i, stats_ref, acc,
                   jnp.sum(t, axis=0, keepdims=True),
                   jnp.sum(t * t, axis=0, keepdims=True))

    return pl.pallas_call(
        body,
        grid=(NK // 8 // RB8,),
        in_specs=[
            pl.BlockSpec((RB8, 128), lambda i: (i, 0)),
            pl.BlockSpec((128, 32), lambda i: (0, 0)),
        ],
        out_specs=pl.BlockSpec((2, 32), lambda i: (0, 0)),
        out_shape=jax.ShapeDtypeStruct((2, 32), jnp.float32),
        scratch_shapes=[pltpu.VMEM((2, 32), jnp.float32)],
    )(y8, wp1big8)


def _rqk_block(g_ref, y_ref, fp_ref, wq_ref, wk_ref, wp1_ref, wp2_ref,
               cv_ref, c8_ref):
    a1 = cv_ref[0:1, :]
    b1 = cv_ref[1:2, :]
    bq = cv_ref[2:3, :]
    bk = cv_ref[3:4, :]
    bp2 = cv_ref[4:5, :]
    a2 = c8_ref[0:1, :]
    b2 = c8_ref[1:2, :]

    f2 = jnp.maximum(fp_ref[...] * a1 + b1, 0.0)
    q2 = jnp.dot(f2, wq_ref[...], preferred_element_type=jnp.float32) + bq
    qb = jnp.broadcast_to(q2[None], (K, PN // 2, 2 * D)).reshape(PR2, 2 * D)
    fg = jnp.maximum(g_ref[...] * a1 + b1, 0.0)
    fg = fg.reshape(PR2, 2 * D)
    kg = jnp.dot(fg, wk_ref[...], preferred_element_type=jnp.float32) + bk
    y = y_ref[...].reshape(PR2, 32)
    t = jnp.dot(y, wp1_ref[...], preferred_element_type=jnp.float32)
    t = jnp.maximum(t * a2 + b2, 0.0)
    pr = jnp.dot(t, wp2_ref[...], preferred_element_type=jnp.float32) + bp2
    r = kg - qb + pr
    return r, fg, pr


_GATHER_SPECS = [
    pl.BlockSpec((K, PN // 2, 2 * D), lambda i: (0, i, 0)),
    pl.BlockSpec((K, PN // 2, 32), lambda i: (0, i, 0)),
    pl.BlockSpec((PN // 2, 2 * D), lambda i: (i, 0)),
    pl.BlockSpec((2 * D, 2 * D), lambda i: (0, 0)),
    pl.BlockSpec((2 * D, 2 * D), lambda i: (0, 0)),
    pl.BlockSpec((32, 8), lambda i: (0, 0)),
    pl.BlockSpec((8, 2 * D), lambda i: (0, 0)),
    pl.BlockSpec((8, 2 * D), lambda i: (0, 0)),
    pl.BlockSpec((2, 8), lambda i: (0, 0)),
]


def _pass_b(g2, y2, fpre, wqt, wkbig, wp1big, wp2big, cv, c8, wvbig):
    def body(g_ref, y_ref, fpre_ref, wq_ref, wk_ref, wp1_ref, wp2_ref, cv_ref,
             c8_ref, wv_ref, rb_ref, svb_ref, stats_ref, acc):
        i = pl.program_id(0)
        r, fg, pr = _rqk_block(g_ref, y_ref, fpre_ref, wq_ref, wk_ref, wp1_ref,
                               wp2_ref, cv_ref, c8_ref)
        bv = cv_ref[5:6, :]
        vg = jnp.dot(fg, wv_ref[...], preferred_element_type=jnp.float32) + bv
        rb_ref[...] = r.reshape(K, PN // 2, 2 * D).astype(jnp.bfloat16)
        svb_ref[...] = (vg + pr).reshape(K, PN // 2, 2 * D).astype(jnp.bfloat16)
        _acc_stats(i, stats_ref, acc,
                   jnp.sum(r, axis=0, keepdims=True),
                   jnp.sum(r * r, axis=0, keepdims=True))

    return pl.pallas_call(
        body,
        grid=(N // PN,),
        in_specs=_GATHER_SPECS + [
            pl.BlockSpec((2 * D, 2 * D), lambda i: (0, 0)),
        ],
        out_specs=[
            pl.BlockSpec((K, PN // 2, 2 * D), lambda i: (0, i, 0)),
            pl.BlockSpec((K, PN // 2, 2 * D), lambda i: (0, i, 0)),
            pl.BlockSpec((2, 2 * D), lambda i: (0, 0)),
        ],
        out_shape=[
            jax.ShapeDtypeStruct((K, N // 2, 2 * D), jnp.bfloat16),
            jax.ShapeDtypeStruct((K, N // 2, 2 * D), jnp.bfloat16),
            jax.ShapeDtypeStruct((2, 2 * D), jnp.float32),
        ],
        scratch_shapes=[pltpu.VMEM((2, 2 * D), jnp.float32)],
    )(g2, y2, fpre, wqt, wkbig, wp1big, wp2big, cv, c8, wvbig)


def _pass_c(rb, ww1big, c3):
    def body(rb_ref, ww1_ref, c3_ref, stats_ref, acc):
        i = pl.program_id(0)
        r = rb_ref[...].reshape(PR2, 2 * D).astype(jnp.float32)
        u = _lrelu(r * c3_ref[0:1, :] + c3_ref[1:2, :])
        w1 = jnp.dot(u, ww1_ref[...], preferred_element_type=jnp.float32)
        _acc_stats(i, stats_ref, acc,
                   jnp.sum(w1, axis=0, keepdims=True),
                   jnp.sum(w1 * w1, axis=0, keepdims=True))

    return pl.pallas_call(
        body,
        grid=(N // PN,),
        in_specs=[
            pl.BlockSpec((K, PN // 2, 2 * D), lambda i: (0, i, 0)),
            pl.BlockSpec((2 * D, 16), lambda i: (0, 0)),
            pl.BlockSpec((2, 2 * D), lambda i: (0, 0)),
        ],
        out_specs=pl.BlockSpec((2, 16), lambda i: (0, 0)),
        out_shape=jax.ShapeDtypeStruct((2, 16), jnp.float32),
        scratch_shapes=[pltpu.VMEM((2, 16), jnp.float32)],
    )(rb, ww1big, c3)


def _pass_d(rb, svb, ww1big, c3, ww2big, c16, sel16):
    def body(rb_ref, svb_ref, ww1_ref, c3_ref, ww2_ref, c16_ref, sel16_ref,
             x_ref, stats_ref, acc):
        i = pl.program_id(0)
        r = rb_ref[...].reshape(PR2, 2 * D).astype(jnp.float32)
        u = _lrelu(r * c3_ref[0:1, :] + c3_ref[1:2, :])
        w1 = jnp.dot(u, ww1_ref[...], preferred_element_type=jnp.float32)
        u4 = jnp.maximum(w1 * c16_ref[0:1, :] + c16_ref[1:2, :], 0.0)
        w2 = jnp.dot(u4, ww2_ref[...], preferred_element_type=jnp.float32)
        w2 = w2 + c16_ref[2:3, :]
        e = jnp.exp(w2)
        efull = jnp.dot(e, sel16_ref[...],
                        preferred_element_type=jnp.float32,
                        precision=jax.lax.Precision.HIGHEST)
        sv = svb_ref[...].reshape(PR2, 2 * D).astype(jnp.float32)
        p3 = (sv * efull).reshape(K, PN // 2, 2 * D)
        numer = jnp.sum(p3, axis=0)
        zf = jnp.sum(efull.reshape(K, PN // 2, 2 * D), axis=0)
        x = numer / zf
        x_ref[...] = x
        _acc_stats(i, stats_ref, acc,
                   jnp.sum(x, axis=0, keepdims=True),
                   jnp.sum(x * x, axis=0, keepdims=True))

    return pl.pallas_call(
        body,
        grid=(N // PN,),
        in_specs=[
            pl.BlockSpec((K, PN // 2, 2 * D), lambda i: (0, i, 0)),
            pl.BlockSpec((K, PN // 2, 2 * D), lambda i: (0, i, 0)),
            pl.BlockSpec((2 * D, 16), lambda i: (0, 0)),
            pl.BlockSpec((2, 2 * D), lambda i: (0, 0)),
            pl.BlockSpec((16, 16), lambda i: (0, 0)),
            pl.BlockSpec((4, 16), lambda i: (0, 0)),
            pl.BlockSpec((16, 2 * D), lambda i: (0, 0)),
        ],
        out_specs=[
            pl.BlockSpec((PN // 2, 2 * D), lambda i: (i, 0)),
            pl.BlockSpec((2, 2 * D), lambda i: (0, 0)),
        ],
        out_shape=[
            jax.ShapeDtypeStruct((N // 2, 2 * D), jnp.float32),
            jax.ShapeDtypeStruct((2, 2 * D), jnp.float32),
        ],
        scratch_shapes=[pltpu.VMEM((2, 2 * D), jnp.float32)],
    )(rb, svb, ww1big, c3, ww2big, c16, sel16)


def _pass_e2(xagg2, wc2big, c5):
    def body(x_ref, w_ref, c_ref, x2_ref, stats_ref, acc):
        i = pl.program_id(0)
        u = _lrelu(x_ref[...] * c_ref[0:1, :] + c_ref[1:2, :])
        x2 = jnp.dot(u, w_ref[...], preferred_element_type=jnp.float32)
        x2_ref[...] = x2
        _acc_stats(i, stats_ref, acc,
                   jnp.sum(x2, axis=0, keepdims=True),
                   jnp.sum(x2 * x2, axis=0, keepdims=True))

    return pl.pallas_call(
        body,
        grid=(N // 2 // BN2,),
        in_specs=[
            pl.BlockSpec((BN2, 2 * D), lambda i: (i, 0)),
            pl.BlockSpec((2 * D, 2 * D), lambda i: (0, 0)),
            pl.BlockSpec((2, 2 * D), lambda i: (0, 0)),
        ],
        out_specs=[
            pl.BlockSpec((BN2, 2 * D), lambda i: (i, 0)),
            pl.BlockSpec((2, 2 * D), lambda i: (0, 0)),
        ],
        out_shape=[
            jax.ShapeDtypeStruct((N // 2, 2 * D), jnp.float32),
            jax.ShapeDtypeStruct((2, 2 * D), jnp.float32),
        ],
        scratch_shapes=[pltpu.VMEM((2, 2 * D), jnp.float32)],
    )(xagg2, wc2big, c5)


def _pass_e3(fpre2, x22, ce):
    def body(fpre_ref, x2_ref, c_ref, out_ref):
        f = jnp.maximum(fpre_ref[...] * c_ref[0:1, :] + c_ref[1:2, :], 0.0)
        xb = x2_ref[...] * c_ref[2:3, :] + c_ref[3:4, :]
        out_ref[...] = _lrelu(f + xb)

    return pl.pallas_call(
        body,
        grid=(N // 2 // BN2,),
        in_specs=[
            pl.BlockSpec((BN2, 2 * D), lambda i: (i, 0)),
            pl.BlockSpec((BN2, 2 * D), lambda i: (i, 0)),
            pl.BlockSpec((4, 2 * D), lambda i: (0, 0)),
        ],
        out_specs=pl.BlockSpec((BN2, 2 * D), lambda i: (i, 0)),
        out_shape=jax.ShapeDtypeStruct((N // 2, 2 * D), jnp.float32),
    )(fpre2, x22, ce)


def _bn_affine(g, b, s1, s2, m):
    mean = s1 / m
    var = s2 / m - mean * mean
    a = g / jnp.sqrt(var + EPS)
    return a, b - mean * a


def kernel(feature, xyz, params, neigh_idx):
    p = params
    x2 = feature[0, :, :, 0].T.reshape(N // 2, 2 * D)
    xyzp = jnp.pad(xyz[0], ((0, 0), (0, 13)))
    idx = neigh_idx[0].T.reshape(-1).astype(jnp.int32)

    fpre2, st1r = _pass_a(x2, _bd2(p['W1'].T))
    st1 = st1r[:, :D] + st1r[:, D:]
    a1, b1 = _bn_affine(p['g1'], p['b1'], st1[0], st1[1], N)

    g_flat, y_flat = _sc_gather(fpre2.reshape(N, D), xyzp, idx)
    g2 = g_flat.reshape(K, N // 2, 2 * D)
    y2 = y_flat.reshape(K, N // 2, 32)
    y8 = y_flat.reshape(NK // 8, 128)

    wp1t16 = jnp.zeros((16, 4), jnp.float32).at[:3, :3].set(p['Wp1'].T)
    wp1big8 = jax.scipy.linalg.block_diag(*([wp1t16] * 8))
    st2r = _pass_b0(y8, wp1big8)
    st2 = st2r.reshape(2, 8, 4).sum(axis=1)
    g2p = jnp.pad(p['gp1'], (0, 1))
    b2p = jnp.pad(p['bp1'], (0, 1))
    a2, b2 = _bn_affine(g2p, b2p, st2[0], st2[1], NK)

    cv = jnp.stack([_dup(a1), _dup(b1),
                    _dup(p['bq']),
                    _dup(p['bk']), _dup(p['bp2']), _dup(p['bv']),
                    jnp.zeros(2 * D, jnp.float32),
                    jnp.zeros(2 * D, jnp.float32)])
    c8 = jnp.stack([_dup(a2), _dup(b2)])
    wqt = _bd2(p['Wq'].T)
    wkbig = _bd2(p['Wk'].T)
    wp1big = _bd2(wp1t16)
    wp2big = _bd2(jnp.pad(p['Wp2'].T, ((0, 1), (0, 0))))

    rb, svb, st3r = _pass_b(g2, y2, fpre2, wqt, wkbig, wp1big, wp2big, cv, c8,
                            _bd2(p['Wv'].T))
    st3 = st3r[:, :D] + st3r[:, D:]
    a3, b3 = _bn_affine(p['gw0'], p['bw0'], st3[0], st3[1], NK)
    c3 = jnp.stack([_dup(a3), _dup(b3)])

    ww1big = _bd2(p['Ww1'].T)
    st4r = _pass_c(rb, ww1big, c3)
    st4 = st4r[:, :8] + st4r[:, 8:]
    a4, b4 = _bn_affine(p['gw1'], p['bw1'], st4[0], st4[1], NK)

    c16 = jnp.stack([_dup(a4), _dup(b4), _dup(p['bw2']),
                     jnp.zeros(16, jnp.float32)])
    lanes = jnp.arange(2 * D)
    sel16 = (jnp.arange(16)[:, None]
             == jnp.where(lanes < D, lanes % 8, 8 + lanes % 8)[None, :]
             ).astype(jnp.float32)
    xagg2, st5r = _pass_d(rb, svb, ww1big, c3, _bd2(p['Ww2'].T), c16, sel16)
    st5 = st5r[:, :D] + st5r[:, D:]
    a5, b5 = _bn_affine(p['g_bn'], p['b_bn'], st5[0], st5[1], N)

    c5 = jnp.stack([_dup(a5), _dup(b5)])
    x22, st6r = _pass_e2(xagg2, _bd2(p['Wc2'].T), c5)
    st6 = st6r[:, :D] + st6r[:, D:]
    a6, b6 = _bn_affine(p['gc2'], p['bc2'], st6[0], st6[1], N)

    ce = jnp.stack([_dup(a1), _dup(b1), _dup(a6), _dup(b6)])
    out = _pass_e3(fpre2, x22, ce).reshape(N, D)
    return out.T[None, :, :, None]


# speedup vs baseline: 1.4150x; 1.0035x over previous
import functools

import jax
import jax.numpy as jnp
from jax import lax
from jax.experimental import pallas as pl
from jax.experimental.pallas import tpu as pltpu
from jax.experimental.pallas import tpu_sc as plsc

N = 50000
K = 16
D = 64
NK = N * K
EPS = 1e-5

BN2 = 1000
PN = 400
PR2 = PN * K // 2
RB8 = 4000
CHUNK = 128
NW = 32
NCHUNKS = NK // CHUNK


def _lrelu(x):
    return jnp.maximum(x, 0.2 * x)


def _dup(v):
    return jnp.concatenate([v, v])


def _bd2(w):
    a, b = w.shape
    z = jnp.zeros((2 * a, 2 * b), w.dtype)
    return z.at[:a, :b].set(w).at[a:, b:].set(w)


def _sc_gather(fpre, xyzp, idx):
    mesh = plsc.VectorSubcoreMesh(core_axis_name="c", subcore_axis_name="s")
    nj = (NCHUNKS + NW - 1) // NW

    @functools.partial(
        pl.kernel,
        mesh=mesh,
        compiler_params=pltpu.CompilerParams(use_tc_tiling_on_sc=False),
        out_type=[
            jax.ShapeDtypeStruct((NK, D), jnp.float32),
            jax.ShapeDtypeStruct((NK, 16), jnp.float32),
        ],
        scratch_types=[
            pltpu.VMEM((CHUNK,), jnp.int32),
            pltpu.VMEM((CHUNK,), jnp.int32),
            pltpu.VMEM((CHUNK, D), jnp.float32),
            pltpu.VMEM((CHUNK, D), jnp.float32),
            pltpu.VMEM((CHUNK, 16), jnp.float32),
            pltpu.VMEM((CHUNK, 16), jnp.float32),
            pltpu.SemaphoreType.DMA,
            pltpu.SemaphoreType.DMA,
            pltpu.SemaphoreType.DMA,
            pltpu.SemaphoreType.DMA,
        ],
    )
    def k(fpre_hbm, xyzp_hbm, idx_hbm, g_hbm, y_hbm,
          idx0, idx1, rows0, rows1, yrows0, yrows1,
          gsem0, gsem1, ysem0, ysem1):
        wid = lax.axis_index("s") * 2 + lax.axis_index("c")
        idxs = (idx0, idx1)
        rows = (rows0, rows1)
        yrows = (yrows0, yrows1)
        gsems = (gsem0, gsem1)
        ysems = (ysem0, ysem1)

        def issue(j, b):
            c = j * NW + wid

            @pl.when(c < NCHUNKS)
            def _():
                base = c * CHUNK
                pltpu.sync_copy(idx_hbm.at[pl.ds(base, CHUNK)], idxs[b])
                pltpu.async_copy(fpre_hbm.at[idxs[b]], rows[b], gsems[b])
                pltpu.async_copy(xyzp_hbm.at[idxs[b]], yrows[b], ysems[b])

        def drain(j, b):
            c = j * NW + wid

            @pl.when(c < NCHUNKS)
            def _():
                base = c * CHUNK
                pltpu.make_async_copy(fpre_hbm.at[idxs[b]], rows[b],
                                      gsems[b]).wait()
                pltpu.make_async_copy(xyzp_hbm.at[idxs[b]], yrows[b],
                                      ysems[b]).wait()
                pltpu.sync_copy(rows[b], g_hbm.at[pl.ds(base, CHUNK)])
                pltpu.sync_copy(yrows[b], y_hbm.at[pl.ds(base, CHUNK)])

        issue(0, 0)

        def body(j2, carry):
            j0 = j2 * 2
            issue(j0 + 1, 1)
            drain(j0, 0)
            issue(j0 + 2, 0)
            drain(j0 + 1, 1)
            return carry

        lax.fori_loop(0, nj // 2, body, 0)

    return k(fpre, xyzp, idx)


def _acc_stats(i, stats_ref, acc, s1, s2):
    @pl.when(i == 0)
    def _():
        acc[...] = jnp.zeros_like(acc)

    acc[0:1, :] += s1
    acc[1:2, :] += s2

    @pl.when(i == pl.num_programs(0) - 1)
    def _():
        stats_ref[...] = acc[...]


def _pass_a(x2, w1big):
    def body(x_ref, w_ref, fpre_ref, stats_ref, acc):
        i = pl.program_id(0)
        t = jnp.dot(x_ref[...], w_ref[...], preferred_element_type=jnp.float32)
        fpre_ref[...] = t
        _acc_stats(i, stats_ref, acc,
                   jnp.sum(t, axis=0, keepdims=True),
                   jnp.sum(t * t, axis=0, keepdims=True))

    return pl.pallas_call(
        body,
        grid=(N // 2 // BN2,),
        in_specs=[
            pl.BlockSpec((BN2, 2 * D), lambda i: (i, 0)),
            pl.BlockSpec((2 * D, 2 * D), lambda i: (0, 0)),
        ],
        out_specs=[
            pl.BlockSpec((BN2, 2 * D), lambda i: (i, 0)),
            pl.BlockSpec((2, 2 * D), lambda i: (0, 0)),
        ],
        out_shape=[
            jax.ShapeDtypeStruct((N // 2, 2 * D), jnp.float32),
            jax.ShapeDtypeStruct((2, 2 * D), jnp.float32),
        ],
        scratch_shapes=[pltpu.VMEM((2, 2 * D), jnp.float32)],
    )(x2, w1big)


def _pass_b0(y8, wp1big8):
    def body(y_ref, w_ref, stats_ref, acc):
        i = pl.program_id(0)
        t = jnp.dot(y_ref[...], w_ref[...], preferred_element_type=jnp.float32)
        _acc_stats(i, stats_ref, acc,
                   jnp.sum(t, axis=0, keepdims=True),
                   jnp.sum(t * t, axis=0, keepdims=True))

    return pl.pallas_call(
        body,
        grid=(NK // 8 // RB8,),
        in_specs=[
            pl.BlockSpec((RB8, 128), lambda i: (i, 0)),
            pl.BlockSpec((128, 32), lambda i: (0, 0)),
        ],
        out_specs=pl.BlockSpec((2, 32), lambda i: (0, 0)),
        out_shape=jax.ShapeDtypeStruct((2, 32), jnp.float32),
        scratch_shapes=[pltpu.VMEM((2, 32), jnp.float32)],
    )(y8, wp1big8)


def _rqk_block(g_ref, y_ref, fp_ref, wq_ref, wk_ref, wp1_ref, wp2_ref,
               cv_ref, c8_ref):
    a1 = cv_ref[0:1, :]
    b1 = cv_ref[1:2, :]
    bq = cv_ref[2:3, :]
    bk = cv_ref[3:4, :]
    bp2 = cv_ref[4:5, :]
    a2 = c8_ref[0:1, :]
    b2 = c8_ref[1:2, :]

    f2 = jnp.maximum(fp_ref[...] * a1 + b1, 0.0)
    q2 = jnp.dot(f2, wq_ref[...], preferred_element_type=jnp.float32) + bq
    qb = jnp.broadcast_to(q2[None], (K, PN // 2, 2 * D)).reshape(PR2, 2 * D)
    fg = jnp.maximum(g_ref[...] * a1 + b1, 0.0)
    fg = fg.reshape(PR2, 2 * D)
    kg = jnp.dot(fg, wk_ref[...], preferred_element_type=jnp.float32) + bk
    y = y_ref[...].reshape(PR2, 32)
    t = jnp.dot(y, wp1_ref[...], preferred_element_type=jnp.float32)
    t = jnp.maximum(t * a2 + b2, 0.0)
    pr = jnp.dot(t, wp2_ref[...], preferred_element_type=jnp.float32) + bp2
    r = kg - qb + pr
    return r, fg, pr


_GATHER_SPECS = [
    pl.BlockSpec((K, PN // 2, 2 * D), lambda i: (0, i, 0)),
    pl.BlockSpec((K, PN // 2, 32), lambda i: (0, i, 0)),
    pl.BlockSpec((PN // 2, 2 * D), lambda i: (i, 0)),
    pl.BlockSpec((2 * D, 2 * D), lambda i: (0, 0)),
    pl.BlockSpec((2 * D, 2 * D), lambda i: (0, 0)),
    pl.BlockSpec((32, 8), lambda i: (0, 0)),
    pl.BlockSpec((8, 2 * D), lambda i: (0, 0)),
    pl.BlockSpec((8, 2 * D), lambda i: (0, 0)),
    pl.BlockSpec((2, 8), lambda i: (0, 0)),
]


def _pass_b(g2, y2, fpre, wqt, wkbig, wp1big, wp2big, cv, c8, wvbig):
    def body(g_ref, y_ref, fpre_ref, wq_ref, wk_ref, wp1_ref, wp2_ref, cv_ref,
             c8_ref, wv_ref, rb_ref, svb_ref, stats_ref, acc):
        i = pl.program_id(0)
        r, fg, pr = _rqk_block(g_ref, y_ref, fpre_ref, wq_ref, wk_ref, wp1_ref,
                               wp2_ref, cv_ref, c8_ref)
        bv = cv_ref[5:6, :]
        vg = jnp.dot(fg, wv_ref[...], preferred_element_type=jnp.float32) + bv
        rb_ref[...] = r.reshape(K, PN // 2, 2 * D).astype(jnp.bfloat16)
        svb_ref[...] = (vg + pr).reshape(K, PN // 2, 2 * D).astype(jnp.bfloat16)
        _acc_stats(i, stats_ref, acc,
                   jnp.sum(r, axis=0, keepdims=True),
                   jnp.sum(r * r, axis=0, keepdims=True))

    return pl.pallas_call(
        body,
        grid=(N // PN,),
        in_specs=_GATHER_SPECS + [
            pl.BlockSpec((2 * D, 2 * D), lambda i: (0, 0)),
        ],
        out_specs=[
            pl.BlockSpec((K, PN // 2, 2 * D), lambda i: (0, i, 0)),
            pl.BlockSpec((K, PN // 2, 2 * D), lambda i: (0, i, 0)),
            pl.BlockSpec((2, 2 * D), lambda i: (0, 0)),
        ],
        out_shape=[
            jax.ShapeDtypeStruct((K, N // 2, 2 * D), jnp.bfloat16),
            jax.ShapeDtypeStruct((K, N // 2, 2 * D), jnp.bfloat16),
            jax.ShapeDtypeStruct((2, 2 * D), jnp.float32),
        ],
        scratch_shapes=[pltpu.VMEM((2, 2 * D), jnp.float32)],
    )(g2, y2, fpre, wqt, wkbig, wp1big, wp2big, cv, c8, wvbig)


def _pass_c(rb, ww1big, c3):
    def body(rb_ref, ww1_ref, c3_ref, stats_ref, acc):
        i = pl.program_id(0)
        r = rb_ref[...].reshape(PR2, 2 * D).astype(jnp.float32)
        u = _lrelu(r * c3_ref[0:1, :] + c3_ref[1:2, :])
        w1 = jnp.dot(u, ww1_ref[...], preferred_element_type=jnp.float32)
        _acc_stats(i, stats_ref, acc,
                   jnp.sum(w1, axis=0, keepdims=True),
                   jnp.sum(w1 * w1, axis=0, keepdims=True))

    return pl.pallas_call(
        body,
        grid=(N // PN,),
        in_specs=[
            pl.BlockSpec((K, PN // 2, 2 * D), lambda i: (0, i, 0)),
            pl.BlockSpec((2 * D, 16), lambda i: (0, 0)),
            pl.BlockSpec((2, 2 * D), lambda i: (0, 0)),
        ],
        out_specs=pl.BlockSpec((2, 16), lambda i: (0, 0)),
        out_shape=jax.ShapeDtypeStruct((2, 16), jnp.float32),
        scratch_shapes=[pltpu.VMEM((2, 16), jnp.float32)],
    )(rb, ww1big, c3)


def _pass_d(rb, svb, ww1big, c3, ww2big, c16, sel16):
    def body(rb_ref, svb_ref, ww1_ref, c3_ref, ww2_ref, c16_ref, sel16_ref,
             x_ref, stats_ref, acc):
        i = pl.program_id(0)
        r = rb_ref[...].reshape(PR2, 2 * D).astype(jnp.float32)
        u = _lrelu(r * c3_ref[0:1, :] + c3_ref[1:2, :])
        w1 = jnp.dot(u, ww1_ref[...], preferred_element_type=jnp.float32)
        u4 = jnp.maximum(w1 * c16_ref[0:1, :] + c16_ref[1:2, :], 0.0)
        w2 = jnp.dot(u4, ww2_ref[...], preferred_element_type=jnp.float32)
        w2 = w2 + c16_ref[2:3, :]
        e = jnp.exp(w2)
        efull = jnp.dot(e, sel16_ref[...],
                        preferred_element_type=jnp.float32,
                        precision=jax.lax.Precision.HIGHEST)
        sv = svb_ref[...].reshape(PR2, 2 * D).astype(jnp.float32)
        p3 = (sv * efull).reshape(K, PN // 2, 2 * D)
        numer = jnp.sum(p3, axis=0)
        zf = jnp.sum(efull.reshape(K, PN // 2, 2 * D), axis=0)
        x = numer / zf
        x_ref[...] = x
        _acc_stats(i, stats_ref, acc,
                   jnp.sum(x, axis=0, keepdims=True),
                   jnp.sum(x * x, axis=0, keepdims=True))

    return pl.pallas_call(
        body,
        grid=(N // PN,),
        in_specs=[
            pl.BlockSpec((K, PN // 2, 2 * D), lambda i: (0, i, 0)),
            pl.BlockSpec((K, PN // 2, 2 * D), lambda i: (0, i, 0)),
            pl.BlockSpec((2 * D, 16), lambda i: (0, 0)),
            pl.BlockSpec((2, 2 * D), lambda i: (0, 0)),
            pl.BlockSpec((16, 16), lambda i: (0, 0)),
            pl.BlockSpec((4, 16), lambda i: (0, 0)),
            pl.BlockSpec((16, 2 * D), lambda i: (0, 0)),
        ],
        out_specs=[
            pl.BlockSpec((PN // 2, 2 * D), lambda i: (i, 0)),
            pl.BlockSpec((2, 2 * D), lambda i: (0, 0)),
        ],
        out_shape=[
            jax.ShapeDtypeStruct((N // 2, 2 * D), jnp.float32),
            jax.ShapeDtypeStruct((2, 2 * D), jnp.float32),
        ],
        scratch_shapes=[pltpu.VMEM((2, 2 * D), jnp.float32)],
    )(rb, svb, ww1big, c3, ww2big, c16, sel16)


def _pass_e2(xagg2, wc2big, c5):
    def body(x_ref, w_ref, c_ref, x2_ref, stats_ref, acc):
        i = pl.program_id(0)
        u = _lrelu(x_ref[...] * c_ref[0:1, :] + c_ref[1:2, :])
        x2 = jnp.dot(u, w_ref[...], preferred_element_type=jnp.float32)
        x2_ref[...] = x2
        _acc_stats(i, stats_ref, acc,
                   jnp.sum(x2, axis=0, keepdims=True),
                   jnp.sum(x2 * x2, axis=0, keepdims=True))

    return pl.pallas_call(
        body,
        grid=(N // 2 // BN2,),
        in_specs=[
            pl.BlockSpec((BN2, 2 * D), lambda i: (i, 0)),
            pl.BlockSpec((2 * D, 2 * D), lambda i: (0, 0)),
            pl.BlockSpec((2, 2 * D), lambda i: (0, 0)),
        ],
        out_specs=[
            pl.BlockSpec((BN2, 2 * D), lambda i: (i, 0)),
            pl.BlockSpec((2, 2 * D), lambda i: (0, 0)),
        ],
        out_shape=[
            jax.ShapeDtypeStruct((N // 2, 2 * D), jnp.float32),
            jax.ShapeDtypeStruct((2, 2 * D), jnp.float32),
        ],
        scratch_shapes=[pltpu.VMEM((2, 2 * D), jnp.float32)],
    )(xagg2, wc2big, c5)


def _pass_e3(fpre2, x22, ce):
    def body(fpre_ref, x2_ref, c_ref, out_ref):
        f = jnp.maximum(fpre_ref[...] * c_ref[0:1, :] + c_ref[1:2, :], 0.0)
        xb = x2_ref[...] * c_ref[2:3, :] + c_ref[3:4, :]
        out_ref[...] = _lrelu(f + xb)

    return pl.pallas_call(
        body,
        grid=(N // 2 // BN2,),
        in_specs=[
            pl.BlockSpec((BN2, 2 * D), lambda i: (i, 0)),
            pl.BlockSpec((BN2, 2 * D), lambda i: (i, 0)),
            pl.BlockSpec((4, 2 * D), lambda i: (0, 0)),
        ],
        out_specs=pl.BlockSpec((BN2, 2 * D), lambda i: (i, 0)),
        out_shape=jax.ShapeDtypeStruct((N // 2, 2 * D), jnp.float32),
    )(fpre2, x22, ce)


def _bn_affine(g, b, s1, s2, m):
    mean = s1 / m
    var = s2 / m - mean * mean
    a = g / jnp.sqrt(var + EPS)
    return a, b - mean * a


def kernel(feature, xyz, params, neigh_idx):
    p = params
    x2 = feature[0, :, :, 0].T.reshape(N // 2, 2 * D)
    xyzp = jnp.pad(xyz[0], ((0, 0), (0, 13)))
    idx = neigh_idx[0].T.reshape(-1).astype(jnp.int32)

    fpre2, st1r = _pass_a(x2, _bd2(p['W1'].T))
    st1 = st1r[:, :D] + st1r[:, D:]
    a1, b1 = _bn_affine(p['g1'], p['b1'], st1[0], st1[1], N)

    g_flat, y_flat = _sc_gather(fpre2.reshape(N, D), xyzp, idx)
    g2 = g_flat.reshape(K, N // 2, 2 * D)
    y2 = y_flat.reshape(K, N // 2, 32)
    y8 = y_flat.reshape(NK // 8, 128)

    wp1t16 = jnp.zeros((16, 4), jnp.float32).at[:3, :3].set(p['Wp1'].T)
    wp1big8 = jax.scipy.linalg.block_diag(*([wp1t16] * 8))
    st2r = _pass_b0(y8, wp1big8)
    st2 = st2r.reshape(2, 8, 4).sum(axis=1)
    g2p = jnp.pad(p['gp1'], (0, 1))
    b2p = jnp.pad(p['bp1'], (0, 1))
    a2, b2 = _bn_affine(g2p, b2p, st2[0], st2[1], NK)

    cv = jnp.stack([_dup(a1), _dup(b1),
                    _dup(p['bq']),
                    _dup(p['bk']), _dup(p['bp2']), _dup(p['bv']),
                    jnp.zeros(2 * D, jnp.float32),
                    jnp.zeros(2 * D, jnp.float32)])
    c8 = jnp.stack([_dup(a2), _dup(b2)])
    wqt = _bd2(p['Wq'].T)
    wkbig = _bd2(p['Wk'].T)
    wp1big = _bd2(wp1t16)
    wp2big = _bd2(jnp.pad(p['Wp2'].T, ((0, 1), (0, 0))))

    rb, svb, st3r = _pass_b(g2, y2, fpre2, wqt, wkbig, wp1big, wp2big, cv, c8,
                            _bd2(p['Wv'].T))
    st3 = st3r[:, :D] + st3r[:, D:]
    a3, b3 = _bn_affine(p['gw0'], p['bw0'], st3[0], st3[1], NK)
    c3 = jnp.stack([_dup(a3), _dup(b3)])

    ww1big = _bd2(p['Ww1'].T)
    st4r = _pass_c(rb, ww1big, c3)
    st4 = st4r[:, :8] + st4r[:, 8:]
    a4, b4 = _bn_affine(p['gw1'], p['bw1'], st4[0], st4[1], NK)

    c16 = jnp.stack([_dup(a4), _dup(b4), _dup(p['bw2']),
                     jnp.zeros(16, jnp.float32)])
    lanes = jnp.arange(2 * D)
    sel16 = (jnp.arange(16)[:, None]
             == jnp.where(lanes < D, lanes % 8, 8 + lanes % 8)[None, :]
             ).astype(jnp.float32)
    xagg2, st5r = _pass_d(rb, svb, ww1big, c3, _bd2(p['Ww2'].T), c16, sel16)
    st5 = st5r[:, :D] + st5r[:, D:]
    a5, b5 = _bn_affine(p['g_bn'], p['b_bn'], st5[0], st5[1], N)

    c5 = jnp.stack([_dup(a5), _dup(b5)])
    x22, st6r = _pass_e2(xagg2, _bd2(p['Wc2'].T), c5)
    st6 = st6r[:, :D] + st6r[:, D:]
    a6, b6 = _bn_affine(p['gc2'], p['bc2'], st6[0], st6[1], N)

    ce = jnp.stack([_dup(a1), _dup(b1), _dup(a6), _dup(b6)])
    out = _pass_e3(fpre2, x22, ce).reshape(N, D)
    return out.T[None, :, :, None]


# speedup vs baseline: 1.4366x; 1.0153x over previous
import functools

import jax
import jax.numpy as jnp
from jax import lax
from jax.experimental import pallas as pl
from jax.experimental.pallas import tpu as pltpu
from jax.experimental.pallas import tpu_sc as plsc

N = 50000
K = 16
D = 64
NK = N * K
EPS = 1e-5

BN2 = 1000
PN = 400
PR2 = PN * K // 2
RB8 = 4000
CHUNK = 128
NW = 32
NCHUNKS = NK // CHUNK


def _lrelu(x):
    return jnp.maximum(x, 0.2 * x)


def _dup(v):
    return jnp.concatenate([v, v])


def _bd2(w):
    a, b = w.shape
    z = jnp.zeros((2 * a, 2 * b), w.dtype)
    return z.at[:a, :b].set(w).at[a:, b:].set(w)


def _sc_row_gather(table, idx, width):
    mesh = plsc.VectorSubcoreMesh(core_axis_name="c", subcore_axis_name="s")
    nj = (NCHUNKS + NW - 1) // NW

    @functools.partial(
        pl.kernel,
        mesh=mesh,
        compiler_params=pltpu.CompilerParams(use_tc_tiling_on_sc=False),
        out_type=jax.ShapeDtypeStruct((NK, width), jnp.float32),
        scratch_types=[
            pltpu.VMEM((CHUNK,), jnp.int32),
            pltpu.VMEM((CHUNK,), jnp.int32),
            pltpu.VMEM((CHUNK, width), jnp.float32),
            pltpu.VMEM((CHUNK, width), jnp.float32),
            pltpu.SemaphoreType.DMA,
            pltpu.SemaphoreType.DMA,
        ],
    )
    def k(tab_hbm, idx_hbm, out_hbm, idx0, idx1, rows0, rows1, sem0, sem1):
        wid = lax.axis_index("s") * 2 + lax.axis_index("c")
        idxs = (idx0, idx1)
        rows = (rows0, rows1)
        sems = (sem0, sem1)

        def issue(j, b):
            c = j * NW + wid

            @pl.when(c < NCHUNKS)
            def _():
                base = c * CHUNK
                pltpu.sync_copy(idx_hbm.at[pl.ds(base, CHUNK)], idxs[b])
                pltpu.async_copy(tab_hbm.at[idxs[b]], rows[b], sems[b])

        def drain(j, b):
            c = j * NW + wid

            @pl.when(c < NCHUNKS)
            def _():
                base = c * CHUNK
                pltpu.make_async_copy(tab_hbm.at[idxs[b]], rows[b],
                                      sems[b]).wait()
                pltpu.sync_copy(rows[b], out_hbm.at[pl.ds(base, CHUNK)])

        issue(0, 0)

        def body(j2, carry):
            j0 = j2 * 2
            issue(j0 + 1, 1)
            drain(j0, 0)
            issue(j0 + 2, 0)
            drain(j0 + 1, 1)
            return carry

        lax.fori_loop(0, nj // 2, body, 0)

    return k(table, idx)


def _acc_stats(i, stats_ref, acc, s1, s2):
    @pl.when(i == 0)
    def _():
        acc[...] = jnp.zeros_like(acc)

    acc[0:1, :] += s1
    acc[1:2, :] += s2

    @pl.when(i == pl.num_programs(0) - 1)
    def _():
        stats_ref[...] = acc[...]


def _pass_a(x2, w1big):
    def body(x_ref, w_ref, fpre_ref, stats_ref, acc):
        i = pl.program_id(0)
        t = jnp.dot(x_ref[...], w_ref[...], preferred_element_type=jnp.float32)
        fpre_ref[...] = t
        _acc_stats(i, stats_ref, acc,
                   jnp.sum(t, axis=0, keepdims=True),
                   jnp.sum(t * t, axis=0, keepdims=True))

    return pl.pallas_call(
        body,
        grid=(N // 2 // BN2,),
        in_specs=[
            pl.BlockSpec((BN2, 2 * D), lambda i: (i, 0)),
            pl.BlockSpec((2 * D, 2 * D), lambda i: (0, 0)),
        ],
        out_specs=[
            pl.BlockSpec((BN2, 2 * D), lambda i: (i, 0)),
            pl.BlockSpec((2, 2 * D), lambda i: (0, 0)),
        ],
        out_shape=[
            jax.ShapeDtypeStruct((N // 2, 2 * D), jnp.float32),
            jax.ShapeDtypeStruct((2, 2 * D), jnp.float32),
        ],
        scratch_shapes=[pltpu.VMEM((2, 2 * D), jnp.float32)],
    )(x2, w1big)


def _pass_b0(y8, wp1big8):
    def body(y_ref, w_ref, stats_ref, acc):
        i = pl.program_id(0)
        t = jnp.dot(y_ref[...], w_ref[...], preferred_element_type=jnp.float32)
        _acc_stats(i, stats_ref, acc,
                   jnp.sum(t, axis=0, keepdims=True),
                   jnp.sum(t * t, axis=0, keepdims=True))

    return pl.pallas_call(
        body,
        grid=(NK // 8 // RB8,),
        in_specs=[
            pl.BlockSpec((RB8, 128), lambda i: (i, 0)),
            pl.BlockSpec((128, 32), lambda i: (0, 0)),
        ],
        out_specs=pl.BlockSpec((2, 32), lambda i: (0, 0)),
        out_shape=jax.ShapeDtypeStruct((2, 32), jnp.float32),
        scratch_shapes=[pltpu.VMEM((2, 32), jnp.float32)],
    )(y8, wp1big8)


def _rqk_block(g_ref, y_ref, fp_ref, wq_ref, wk_ref, wp1_ref, wp2_ref,
               cv_ref, c8_ref):
    a1 = cv_ref[0:1, :]
    b1 = cv_ref[1:2, :]
    bq = cv_ref[2:3, :]
    bk = cv_ref[3:4, :]
    bp2 = cv_ref[4:5, :]
    a2 = c8_ref[0:1, :]
    b2 = c8_ref[1:2, :]

    f2 = jnp.maximum(fp_ref[...] * a1 + b1, 0.0)
    q2 = jnp.dot(f2, wq_ref[...], preferred_element_type=jnp.float32) + bq
    qb = jnp.broadcast_to(q2[None], (K, PN // 2, 2 * D)).reshape(PR2, 2 * D)
    fg = jnp.maximum(g_ref[...] * a1 + b1, 0.0)
    fg = fg.reshape(PR2, 2 * D)
    kg = jnp.dot(fg, wk_ref[...], preferred_element_type=jnp.float32) + bk
    y = y_ref[...].reshape(PR2, 32)
    t = jnp.dot(y, wp1_ref[...], preferred_element_type=jnp.float32)
    t = jnp.maximum(t * a2 + b2, 0.0)
    pr = jnp.dot(t, wp2_ref[...], preferred_element_type=jnp.float32) + bp2
    r = kg - qb + pr
    return r, fg, pr


_GATHER_SPECS = [
    pl.BlockSpec((K, PN // 2, 2 * D), lambda i: (0, i, 0)),
    pl.BlockSpec((K, PN // 2, 32), lambda i: (0, i, 0)),
    pl.BlockSpec((PN // 2, 2 * D), lambda i: (i, 0)),
    pl.BlockSpec((2 * D, 2 * D), lambda i: (0, 0)),
    pl.BlockSpec((2 * D, 2 * D), lambda i: (0, 0)),
    pl.BlockSpec((32, 8), lambda i: (0, 0)),
    pl.BlockSpec((8, 2 * D), lambda i: (0, 0)),
    pl.BlockSpec((8, 2 * D), lambda i: (0, 0)),
    pl.BlockSpec((2, 8), lambda i: (0, 0)),
]


def _pass_b(g2, y2, fpre, wqt, wkbig, wp1big, wp2big, cv, c8, wvbig):
    def body(g_ref, y_ref, fpre_ref, wq_ref, wk_ref, wp1_ref, wp2_ref, cv_ref,
             c8_ref, wv_ref, rb_ref, svb_ref, stats_ref, acc):
        i = pl.program_id(0)
        r, fg, pr = _rqk_block(g_ref, y_ref, fpre_ref, wq_ref, wk_ref, wp1_ref,
                               wp2_ref, cv_ref, c8_ref)
        bv = cv_ref[5:6, :]
        vg = jnp.dot(fg, wv_ref[...], preferred_element_type=jnp.float32) + bv
        rb_ref[...] = r.reshape(K, PN // 2, 2 * D).astype(jnp.bfloat16)
        svb_ref[...] = (vg + pr).reshape(K, PN // 2, 2 * D).astype(jnp.bfloat16)
        _acc_stats(i, stats_ref, acc,
                   jnp.sum(r, axis=0, keepdims=True),
                   jnp.sum(r * r, axis=0, keepdims=True))

    return pl.pallas_call(
        body,
        grid=(N // PN,),
        in_specs=_GATHER_SPECS + [
            pl.BlockSpec((2 * D, 2 * D), lambda i: (0, 0)),
        ],
        out_specs=[
            pl.BlockSpec((K, PN // 2, 2 * D), lambda i: (0, i, 0)),
            pl.BlockSpec((K, PN // 2, 2 * D), lambda i: (0, i, 0)),
            pl.BlockSpec((2, 2 * D), lambda i: (0, 0)),
        ],
        out_shape=[
            jax.ShapeDtypeStruct((K, N // 2, 2 * D), jnp.bfloat16),
            jax.ShapeDtypeStruct((K, N // 2, 2 * D), jnp.bfloat16),
            jax.ShapeDtypeStruct((2, 2 * D), jnp.float32),
        ],
        scratch_shapes=[pltpu.VMEM((2, 2 * D), jnp.float32)],
    )(g2, y2, fpre, wqt, wkbig, wp1big, wp2big, cv, c8, wvbig)


def _pass_c(rb, ww1big, c3):
    def body(rb_ref, ww1_ref, c3_ref, stats_ref, acc):
        i = pl.program_id(0)
        r = rb_ref[...].reshape(PR2, 2 * D).astype(jnp.float32)
        u = _lrelu(r * c3_ref[0:1, :] + c3_ref[1:2, :])
        w1 = jnp.dot(u, ww1_ref[...], preferred_element_type=jnp.float32)
        _acc_stats(i, stats_ref, acc,
                   jnp.sum(w1, axis=0, keepdims=True),
                   jnp.sum(w1 * w1, axis=0, keepdims=True))

    return pl.pallas_call(
        body,
        grid=(N // PN,),
        in_specs=[
            pl.BlockSpec((K, PN // 2, 2 * D), lambda i: (0, i, 0)),
            pl.BlockSpec((2 * D, 16), lambda i: (0, 0)),
            pl.BlockSpec((2, 2 * D), lambda i: (0, 0)),
        ],
        out_specs=pl.BlockSpec((2, 16), lambda i: (0, 0)),
        out_shape=jax.ShapeDtypeStruct((2, 16), jnp.float32),
        scratch_shapes=[pltpu.VMEM((2, 16), jnp.float32)],
    )(rb, ww1big, c3)


def _pass_d(rb, svb, ww1big, c3, ww2big, c16, sel16):
    def body(rb_ref, svb_ref, ww1_ref, c3_ref, ww2_ref, c16_ref, sel16_ref,
             x_ref, stats_ref, acc):
        i = pl.program_id(0)
        r = rb_ref[...].reshape(PR2, 2 * D).astype(jnp.float32)
        u = _lrelu(r * c3_ref[0:1, :] + c3_ref[1:2, :])
        w1 = jnp.dot(u, ww1_ref[...], preferred_element_type=jnp.float32)
        u4 = jnp.maximum(w1 * c16_ref[0:1, :] + c16_ref[1:2, :], 0.0)
        w2 = jnp.dot(u4, ww2_ref[...], preferred_element_type=jnp.float32)
        w2 = w2 + c16_ref[2:3, :]
        e = jnp.exp(w2)
        efull = jnp.dot(e, sel16_ref[...],
                        preferred_element_type=jnp.float32,
                        precision=jax.lax.Precision.HIGHEST)
        sv = svb_ref[...].reshape(PR2, 2 * D).astype(jnp.float32)
        p3 = (sv * efull).reshape(K, PN // 2, 2 * D)
        numer = jnp.sum(p3, axis=0)
        zf = jnp.sum(efull.reshape(K, PN // 2, 2 * D), axis=0)
        x = numer / zf
        x_ref[...] = x
        _acc_stats(i, stats_ref, acc,
                   jnp.sum(x, axis=0, keepdims=True),
                   jnp.sum(x * x, axis=0, keepdims=True))

    return pl.pallas_call(
        body,
        grid=(N // PN,),
        in_specs=[
            pl.BlockSpec((K, PN // 2, 2 * D), lambda i: (0, i, 0)),
            pl.BlockSpec((K, PN // 2, 2 * D), lambda i: (0, i, 0)),
            pl.BlockSpec((2 * D, 16), lambda i: (0, 0)),
            pl.BlockSpec((2, 2 * D), lambda i: (0, 0)),
            pl.BlockSpec((16, 16), lambda i: (0, 0)),
            pl.BlockSpec((4, 16), lambda i: (0, 0)),
            pl.BlockSpec((16, 2 * D), lambda i: (0, 0)),
        ],
        out_specs=[
            pl.BlockSpec((PN // 2, 2 * D), lambda i: (i, 0)),
            pl.BlockSpec((2, 2 * D), lambda i: (0, 0)),
        ],
        out_shape=[
            jax.ShapeDtypeStruct((N // 2, 2 * D), jnp.float32),
            jax.ShapeDtypeStruct((2, 2 * D), jnp.float32),
        ],
        scratch_shapes=[pltpu.VMEM((2, 2 * D), jnp.float32)],
    )(rb, svb, ww1big, c3, ww2big, c16, sel16)


def _pass_e2(xagg2, wc2big, c5):
    def body(x_ref, w_ref, c_ref, x2_ref, stats_ref, acc):
        i = pl.program_id(0)
        u = _lrelu(x_ref[...] * c_ref[0:1, :] + c_ref[1:2, :])
        x2 = jnp.dot(u, w_ref[...], preferred_element_type=jnp.float32)
        x2_ref[...] = x2
        _acc_stats(i, stats_ref, acc,
                   jnp.sum(x2, axis=0, keepdims=True),
                   jnp.sum(x2 * x2, axis=0, keepdims=True))

    return pl.pallas_call(
        body,
        grid=(N // 2 // BN2,),
        in_specs=[
            pl.BlockSpec((BN2, 2 * D), lambda i: (i, 0)),
            pl.BlockSpec((2 * D, 2 * D), lambda i: (0, 0)),
            pl.BlockSpec((2, 2 * D), lambda i: (0, 0)),
        ],
        out_specs=[
            pl.BlockSpec((BN2, 2 * D), lambda i: (i, 0)),
            pl.BlockSpec((2, 2 * D), lambda i: (0, 0)),
        ],
        out_shape=[
            jax.ShapeDtypeStruct((N // 2, 2 * D), jnp.float32),
            jax.ShapeDtypeStruct((2, 2 * D), jnp.float32),
        ],
        scratch_shapes=[pltpu.VMEM((2, 2 * D), jnp.float32)],
    )(xagg2, wc2big, c5)


def _pass_e3(fpre2, x22, ce):
    def body(fpre_ref, x2_ref, c_ref, out_ref):
        f = jnp.maximum(fpre_ref[...] * c_ref[0:1, :] + c_ref[1:2, :], 0.0)
        xb = x2_ref[...] * c_ref[2:3, :] + c_ref[3:4, :]
        out_ref[...] = _lrelu(f + xb)

    return pl.pallas_call(
        body,
        grid=(N // 2 // BN2,),
        in_specs=[
            pl.BlockSpec((BN2, 2 * D), lambda i: (i, 0)),
            pl.BlockSpec((BN2, 2 * D), lambda i: (i, 0)),
            pl.BlockSpec((4, 2 * D), lambda i: (0, 0)),
        ],
        out_specs=pl.BlockSpec((BN2, 2 * D), lambda i: (i, 0)),
        out_shape=jax.ShapeDtypeStruct((N // 2, 2 * D), jnp.float32),
    )(fpre2, x22, ce)


def _bn_affine(g, b, s1, s2, m):
    mean = s1 / m
    var = s2 / m - mean * mean
    a = g / jnp.sqrt(var + EPS)
    return a, b - mean * a


def kernel(feature, xyz, params, neigh_idx):
    p = params
    x2 = feature[0, :, :, 0].T.reshape(N // 2, 2 * D)
    xyzp = jnp.pad(xyz[0], ((0, 0), (0, 13)))
    idx = neigh_idx[0].T.reshape(-1).astype(jnp.int32)

    y_flat = _sc_row_gather(xyzp, idx, 16)
    fpre2, st1r = _pass_a(x2, _bd2(p['W1'].T))
    st1 = st1r[:, :D] + st1r[:, D:]
    a1, b1 = _bn_affine(p['g1'], p['b1'], st1[0], st1[1], N)

    g_flat = _sc_row_gather(fpre2.reshape(N, D), idx, D)
    g2 = g_flat.reshape(K, N // 2, 2 * D)
    y2 = y_flat.reshape(K, N // 2, 32)
    y8 = y_flat.reshape(NK // 8, 128)

    wp1t16 = jnp.zeros((16, 4), jnp.float32).at[:3, :3].set(p['Wp1'].T)
    wp1big8 = jax.scipy.linalg.block_diag(*([wp1t16] * 8))
    st2r = _pass_b0(y8, wp1big8)
    st2 = st2r.reshape(2, 8, 4).sum(axis=1)
    g2p = jnp.pad(p['gp1'], (0, 1))
    b2p = jnp.pad(p['bp1'], (0, 1))
    a2, b2 = _bn_affine(g2p, b2p, st2[0], st2[1], NK)

    cv = jnp.stack([_dup(a1), _dup(b1),
                    _dup(p['bq']),
                    _dup(p['bk']), _dup(p['bp2']), _dup(p['bv']),
                    jnp.zeros(2 * D, jnp.float32),
                    jnp.zeros(2 * D, jnp.float32)])
    c8 = jnp.stack([_dup(a2), _dup(b2)])
    wqt = _bd2(p['Wq'].T)
    wkbig = _bd2(p['Wk'].T)
    wp1big = _bd2(wp1t16)
    wp2big = _bd2(jnp.pad(p['Wp2'].T, ((0, 1), (0, 0))))

    rb, svb, st3r = _pass_b(g2, y2, fpre2, wqt, wkbig, wp1big, wp2big, cv, c8,
                            _bd2(p['Wv'].T))
    st3 = st3r[:, :D] + st3r[:, D:]
    a3, b3 = _bn_affine(p['gw0'], p['bw0'], st3[0], st3[1], NK)
    c3 = jnp.stack([_dup(a3), _dup(b3)])

    ww1big = _bd2(p['Ww1'].T)
    st4r = _pass_c(rb, ww1big, c3)
    st4 = st4r[:, :8] + st4r[:, 8:]
    a4, b4 = _bn_affine(p['gw1'], p['bw1'], st4[0], st4[1], NK)

    c16 = jnp.stack([_dup(a4), _dup(b4), _dup(p['bw2']),
                     jnp.zeros(16, jnp.float32)])
    lanes = jnp.arange(2 * D)
    sel16 = (jnp.arange(16)[:, None]
             == jnp.where(lanes < D, lanes % 8, 8 + lanes % 8)[None, :]
             ).astype(jnp.float32)
    xagg2, st5r = _pass_d(rb, svb, ww1big, c3, _bd2(p['Ww2'].T), c16, sel16)
    st5 = st5r[:, :D] + st5r[:, D:]
    a5, b5 = _bn_affine(p['g_bn'], p['b_bn'], st5[0], st5[1], N)

    c5 = jnp.stack([_dup(a5), _dup(b5)])
    x22, st6r = _pass_e2(xagg2, _bd2(p['Wc2'].T), c5)
    st6 = st6r[:, :D] + st6r[:, D:]
    a6, b6 = _bn_affine(p['gc2'], p['bc2'], st6[0], st6[1], N)

    ce = jnp.stack([_dup(a1), _dup(b1), _dup(a6), _dup(b6)])
    out = _pass_e3(fpre2, x22, ce).reshape(N, D)
    return out.T[None, :, :, None]


# speedup vs baseline: 1.4687x; 1.0224x over previous
import functools

import jax
import jax.numpy as jnp
from jax import lax
from jax.experimental import pallas as pl
from jax.experimental.pallas import tpu as pltpu
from jax.experimental.pallas import tpu_sc as plsc

N = 50000
K = 16
D = 64
NK = N * K
EPS = 1e-5

BN2 = 1000
PN = 400
PR2 = PN * K // 2
RB8 = 4000
CHUNK = 128
NW = 32
NCHUNKS = NK // CHUNK


def _lrelu(x):
    return jnp.maximum(x, 0.2 * x)


def _dup(v):
    return jnp.concatenate([v, v])


def _bd2(w):
    a, b = w.shape
    z = jnp.zeros((2 * a, 2 * b), w.dtype)
    return z.at[:a, :b].set(w).at[a:, b:].set(w)


def _sc_row_gather(table, idx, width):
    mesh = plsc.VectorSubcoreMesh(core_axis_name="c", subcore_axis_name="s")
    nj = (NCHUNKS + NW - 1) // NW
    NB = 4

    @functools.partial(
        pl.kernel,
        mesh=mesh,
        compiler_params=pltpu.CompilerParams(use_tc_tiling_on_sc=False),
        out_type=jax.ShapeDtypeStruct((NK, width), jnp.float32),
        scratch_types=(
            [pltpu.VMEM((CHUNK,), jnp.int32)] * NB
            + [pltpu.VMEM((CHUNK, width), jnp.float32)] * NB
            + [pltpu.SemaphoreType.DMA] * NB
        ),
    )
    def k(tab_hbm, idx_hbm, out_hbm, *bufs):
        idxs = bufs[0:NB]
        rows = bufs[NB:2 * NB]
        sems = bufs[2 * NB:3 * NB]
        wid = lax.axis_index("s") * 2 + lax.axis_index("c")

        def issue(j, b):
            c = j * NW + wid

            @pl.when(c < NCHUNKS)
            def _():
                base = c * CHUNK
                pltpu.sync_copy(idx_hbm.at[pl.ds(base, CHUNK)], idxs[b])
                pltpu.async_copy(tab_hbm.at[idxs[b]], rows[b], sems[b])

        def drain(j, b):
            c = j * NW + wid

            @pl.when(c < NCHUNKS)
            def _():
                base = c * CHUNK
                pltpu.make_async_copy(tab_hbm.at[idxs[b]], rows[b],
                                      sems[b]).wait()
                pltpu.sync_copy(rows[b], out_hbm.at[pl.ds(base, CHUNK)])

        for b in range(NB - 1):
            issue(b, b)

        def body(j4, carry):
            j0 = j4 * NB
            for b in range(NB):
                issue(j0 + b + NB - 1, (b + NB - 1) % NB)
                drain(j0 + b, b)
            return carry

        lax.fori_loop(0, nj // NB, body, 0)

    return k(table, idx)


def _acc_stats(i, stats_ref, acc, s1, s2):
    @pl.when(i == 0)
    def _():
        acc[...] = jnp.zeros_like(acc)

    acc[0:1, :] += s1
    acc[1:2, :] += s2

    @pl.when(i == pl.num_programs(0) - 1)
    def _():
        stats_ref[...] = acc[...]


def _pass_a(x2, w1big):
    def body(x_ref, w_ref, fpre_ref, stats_ref, acc):
        i = pl.program_id(0)
        t = jnp.dot(x_ref[...], w_ref[...], preferred_element_type=jnp.float32)
        fpre_ref[...] = t
        _acc_stats(i, stats_ref, acc,
                   jnp.sum(t, axis=0, keepdims=True),
                   jnp.sum(t * t, axis=0, keepdims=True))

    return pl.pallas_call(
        body,
        grid=(N // 2 // BN2,),
        in_specs=[
            pl.BlockSpec((BN2, 2 * D), lambda i: (i, 0)),
            pl.BlockSpec((2 * D, 2 * D), lambda i: (0, 0)),
        ],
        out_specs=[
            pl.BlockSpec((BN2, 2 * D), lambda i: (i, 0)),
            pl.BlockSpec((2, 2 * D), lambda i: (0, 0)),
        ],
        out_shape=[
            jax.ShapeDtypeStruct((N // 2, 2 * D), jnp.float32),
            jax.ShapeDtypeStruct((2, 2 * D), jnp.float32),
        ],
        scratch_shapes=[pltpu.VMEM((2, 2 * D), jnp.float32)],
    )(x2, w1big)


def _pass_b0(y8, wp1big8):
    def body(y_ref, w_ref, stats_ref, acc):
        i = pl.program_id(0)
        t = jnp.dot(y_ref[...], w_ref[...], preferred_element_type=jnp.float32)
        _acc_stats(i, stats_ref, acc,
                   jnp.sum(t, axis=0, keepdims=True),
                   jnp.sum(t * t, axis=0, keepdims=True))

    return pl.pallas_call(
        body,
        grid=(NK // 8 // RB8,),
        in_specs=[
            pl.BlockSpec((RB8, 128), lambda i: (i, 0)),
            pl.BlockSpec((128, 32), lambda i: (0, 0)),
        ],
        out_specs=pl.BlockSpec((2, 32), lambda i: (0, 0)),
        out_shape=jax.ShapeDtypeStruct((2, 32), jnp.float32),
        scratch_shapes=[pltpu.VMEM((2, 32), jnp.float32)],
    )(y8, wp1big8)


def _rqk_block(g_ref, y_ref, fp_ref, wq_ref, wk_ref, wp1_ref, wp2_ref,
               cv_ref, c8_ref):
    a1 = cv_ref[0:1, :]
    b1 = cv_ref[1:2, :]
    bq = cv_ref[2:3, :]
    bk = cv_ref[3:4, :]
    bp2 = cv_ref[4:5, :]
    a2 = c8_ref[0:1, :]
    b2 = c8_ref[1:2, :]

    f2 = jnp.maximum(fp_ref[...] * a1 + b1, 0.0)
    q2 = jnp.dot(f2, wq_ref[...], preferred_element_type=jnp.float32) + bq
    qb = jnp.broadcast_to(q2[None], (K, PN // 2, 2 * D)).reshape(PR2, 2 * D)
    fg = jnp.maximum(g_ref[...] * a1 + b1, 0.0)
    fg = fg.reshape(PR2, 2 * D)
    kg = jnp.dot(fg, wk_ref[...], preferred_element_type=jnp.float32) + bk
    y = y_ref[...].reshape(PR2, 32)
    t = jnp.dot(y, wp1_ref[...], preferred_element_type=jnp.float32)
    t = jnp.maximum(t * a2 + b2, 0.0)
    pr = jnp.dot(t, wp2_ref[...], preferred_element_type=jnp.float32) + bp2
    r = kg - qb + pr
    return r, fg, pr


_GATHER_SPECS = [
    pl.BlockSpec((K, PN // 2, 2 * D), lambda i: (0, i, 0)),
    pl.BlockSpec((K, PN // 2, 32), lambda i: (0, i, 0)),
    pl.BlockSpec((PN // 2, 2 * D), lambda i: (i, 0)),
    pl.BlockSpec((2 * D, 2 * D), lambda i: (0, 0)),
    pl.BlockSpec((2 * D, 2 * D), lambda i: (0, 0)),
    pl.BlockSpec((32, 8), lambda i: (0, 0)),
    pl.BlockSpec((8, 2 * D), lambda i: (0, 0)),
    pl.BlockSpec((8, 2 * D), lambda i: (0, 0)),
    pl.BlockSpec((2, 8), lambda i: (0, 0)),
]


def _pass_b(g2, y2, fpre, wqt, wkbig, wp1big, wp2big, cv, c8, wvbig):
    def body(g_ref, y_ref, fpre_ref, wq_ref, wk_ref, wp1_ref, wp2_ref, cv_ref,
             c8_ref, wv_ref, rb_ref, svb_ref, stats_ref, acc):
        i = pl.program_id(0)
        r, fg, pr = _rqk_block(g_ref, y_ref, fpre_ref, wq_ref, wk_ref, wp1_ref,
                               wp2_ref, cv_ref, c8_ref)
        bv = cv_ref[5:6, :]
        vg = jnp.dot(fg, wv_ref[...], preferred_element_type=jnp.float32) + bv
        rb_ref[...] = r.reshape(K, PN // 2, 2 * D).astype(jnp.bfloat16)
        svb_ref[...] = (vg + pr).reshape(K, PN // 2, 2 * D).astype(jnp.bfloat16)
        _acc_stats(i, stats_ref, acc,
                   jnp.sum(r, axis=0, keepdims=True),
                   jnp.sum(r * r, axis=0, keepdims=True))

    return pl.pallas_call(
        body,
        grid=(N // PN,),
        in_specs=_GATHER_SPECS + [
            pl.BlockSpec((2 * D, 2 * D), lambda i: (0, 0)),
        ],
        out_specs=[
            pl.BlockSpec((K, PN // 2, 2 * D), lambda i: (0, i, 0)),
            pl.BlockSpec((K, PN // 2, 2 * D), lambda i: (0, i, 0)),
            pl.BlockSpec((2, 2 * D), lambda i: (0, 0)),
        ],
        out_shape=[
            jax.ShapeDtypeStruct((K, N // 2, 2 * D), jnp.bfloat16),
            jax.ShapeDtypeStruct((K, N // 2, 2 * D), jnp.bfloat16),
            jax.ShapeDtypeStruct((2, 2 * D), jnp.float32),
        ],
        scratch_shapes=[pltpu.VMEM((2, 2 * D), jnp.float32)],
    )(g2, y2, fpre, wqt, wkbig, wp1big, wp2big, cv, c8, wvbig)


def _pass_c(rb, ww1big, c3):
    def body(rb_ref, ww1_ref, c3_ref, stats_ref, acc):
        i = pl.program_id(0)
        r = rb_ref[...].reshape(PR2, 2 * D).astype(jnp.float32)
        u = _lrelu(r * c3_ref[0:1, :] + c3_ref[1:2, :])
        w1 = jnp.dot(u, ww1_ref[...], preferred_element_type=jnp.float32)
        _acc_stats(i, stats_ref, acc,
                   jnp.sum(w1, axis=0, keepdims=True),
                   jnp.sum(w1 * w1, axis=0, keepdims=True))

    return pl.pallas_call(
        body,
        grid=(N // PN,),
        in_specs=[
            pl.BlockSpec((K, PN // 2, 2 * D), lambda i: (0, i, 0)),
            pl.BlockSpec((2 * D, 16), lambda i: (0, 0)),
            pl.BlockSpec((2, 2 * D), lambda i: (0, 0)),
        ],
        out_specs=pl.BlockSpec((2, 16), lambda i: (0, 0)),
        out_shape=jax.ShapeDtypeStruct((2, 16), jnp.float32),
        scratch_shapes=[pltpu.VMEM((2, 16), jnp.float32)],
    )(rb, ww1big, c3)


def _pass_d(rb, svb, ww1big, c3, ww2big, c16, sel16):
    def body(rb_ref, svb_ref, ww1_ref, c3_ref, ww2_ref, c16_ref, sel16_ref,
             x_ref, stats_ref, acc):
        i = pl.program_id(0)
        r = rb_ref[...].reshape(PR2, 2 * D).astype(jnp.float32)
        u = _lrelu(r * c3_ref[0:1, :] + c3_ref[1:2, :])
        w1 = jnp.dot(u, ww1_ref[...], preferred_element_type=jnp.float32)
        u4 = jnp.maximum(w1 * c16_ref[0:1, :] + c16_ref[1:2, :], 0.0)
        w2 = jnp.dot(u4, ww2_ref[...], preferred_element_type=jnp.float32)
        w2 = w2 + c16_ref[2:3, :]
        e = jnp.exp(w2)
        efull = jnp.dot(e, sel16_ref[...],
                        preferred_element_type=jnp.float32,
                        precision=jax.lax.Precision.HIGHEST)
        sv = svb_ref[...].reshape(PR2, 2 * D).astype(jnp.float32)
        p3 = (sv * efull).reshape(K, PN // 2, 2 * D)
        numer = jnp.sum(p3, axis=0)
        zf = jnp.sum(efull.reshape(K, PN // 2, 2 * D), axis=0)
        x = numer / zf
        x_ref[...] = x
        _acc_stats(i, stats_ref, acc,
                   jnp.sum(x, axis=0, keepdims=True),
                   jnp.sum(x * x, axis=0, keepdims=True))

    return pl.pallas_call(
        body,
        grid=(N // PN,),
        in_specs=[
            pl.BlockSpec((K, PN // 2, 2 * D), lambda i: (0, i, 0)),
            pl.BlockSpec((K, PN // 2, 2 * D), lambda i: (0, i, 0)),
            pl.BlockSpec((2 * D, 16), lambda i: (0, 0)),
            pl.BlockSpec((2, 2 * D), lambda i: (0, 0)),
            pl.BlockSpec((16, 16), lambda i: (0, 0)),
            pl.BlockSpec((4, 16), lambda i: (0, 0)),
            pl.BlockSpec((16, 2 * D), lambda i: (0, 0)),
        ],
        out_specs=[
            pl.BlockSpec((PN // 2, 2 * D), lambda i: (i, 0)),
            pl.BlockSpec((2, 2 * D), lambda i: (0, 0)),
        ],
        out_shape=[
            jax.ShapeDtypeStruct((N // 2, 2 * D), jnp.float32),
            jax.ShapeDtypeStruct((2, 2 * D), jnp.float32),
        ],
        scratch_shapes=[pltpu.VMEM((2, 2 * D), jnp.float32)],
    )(rb, svb, ww1big, c3, ww2big, c16, sel16)


def _pass_e2(xagg2, wc2big, c5):
    def body(x_ref, w_ref, c_ref, x2_ref, stats_ref, acc):
        i = pl.program_id(0)
        u = _lrelu(x_ref[...] * c_ref[0:1, :] + c_ref[1:2, :])
        x2 = jnp.dot(u, w_ref[...], preferred_element_type=jnp.float32)
        x2_ref[...] = x2
        _acc_stats(i, stats_ref, acc,
                   jnp.sum(x2, axis=0, keepdims=True),
                   jnp.sum(x2 * x2, axis=0, keepdims=True))

    return pl.pallas_call(
        body,
        grid=(N // 2 // BN2,),
        in_specs=[
            pl.BlockSpec((BN2, 2 * D), lambda i: (i, 0)),
            pl.BlockSpec((2 * D, 2 * D), lambda i: (0, 0)),
            pl.BlockSpec((2, 2 * D), lambda i: (0, 0)),
        ],
        out_specs=[
            pl.BlockSpec((BN2, 2 * D), lambda i: (i, 0)),
            pl.BlockSpec((2, 2 * D), lambda i: (0, 0)),
        ],
        out_shape=[
            jax.ShapeDtypeStruct((N // 2, 2 * D), jnp.float32),
            jax.ShapeDtypeStruct((2, 2 * D), jnp.float32),
        ],
        scratch_shapes=[pltpu.VMEM((2, 2 * D), jnp.float32)],
    )(xagg2, wc2big, c5)


def _pass_e3(fpre2, x22, ce):
    def body(fpre_ref, x2_ref, c_ref, out_ref):
        f = jnp.maximum(fpre_ref[...] * c_ref[0:1, :] + c_ref[1:2, :], 0.0)
        xb = x2_ref[...] * c_ref[2:3, :] + c_ref[3:4, :]
        out_ref[...] = _lrelu(f + xb)

    return pl.pallas_call(
        body,
        grid=(N // 2 // BN2,),
        in_specs=[
            pl.BlockSpec((BN2, 2 * D), lambda i: (i, 0)),
            pl.BlockSpec((BN2, 2 * D), lambda i: (i, 0)),
            pl.BlockSpec((4, 2 * D), lambda i: (0, 0)),
        ],
        out_specs=pl.BlockSpec((BN2, 2 * D), lambda i: (i, 0)),
        out_shape=jax.ShapeDtypeStruct((N // 2, 2 * D), jnp.float32),
    )(fpre2, x22, ce)


def _bn_affine(g, b, s1, s2, m):
    mean = s1 / m
    var = s2 / m - mean * mean
    a = g / jnp.sqrt(var + EPS)
    return a, b - mean * a


def kernel(feature, xyz, params, neigh_idx):
    p = params
    x2 = feature[0, :, :, 0].T.reshape(N // 2, 2 * D)
    xyzp = jnp.pad(xyz[0], ((0, 0), (0, 13)))
    idx = neigh_idx[0].T.reshape(-1).astype(jnp.int32)

    y_flat = _sc_row_gather(xyzp, idx, 16)
    fpre2, st1r = _pass_a(x2, _bd2(p['W1'].T))
    st1 = st1r[:, :D] + st1r[:, D:]
    a1, b1 = _bn_affine(p['g1'], p['b1'], st1[0], st1[1], N)

    g_flat = _sc_row_gather(fpre2.reshape(N, D), idx, D)
    g2 = g_flat.reshape(K, N // 2, 2 * D)
    y2 = y_flat.reshape(K, N // 2, 32)
    y8 = y_flat.reshape(NK // 8, 128)

    wp1t16 = jnp.zeros((16, 4), jnp.float32).at[:3, :3].set(p['Wp1'].T)
    wp1big8 = jax.scipy.linalg.block_diag(*([wp1t16] * 8))
    st2r = _pass_b0(y8, wp1big8)
    st2 = st2r.reshape(2, 8, 4).sum(axis=1)
    g2p = jnp.pad(p['gp1'], (0, 1))
    b2p = jnp.pad(p['bp1'], (0, 1))
    a2, b2 = _bn_affine(g2p, b2p, st2[0], st2[1], NK)

    cv = jnp.stack([_dup(a1), _dup(b1),
                    _dup(p['bq']),
                    _dup(p['bk']), _dup(p['bp2']), _dup(p['bv']),
                    jnp.zeros(2 * D, jnp.float32),
                    jnp.zeros(2 * D, jnp.float32)])
    c8 = jnp.stack([_dup(a2), _dup(b2)])
    wqt = _bd2(p['Wq'].T)
    wkbig = _bd2(p['Wk'].T)
    wp1big = _bd2(wp1t16)
    wp2big = _bd2(jnp.pad(p['Wp2'].T, ((0, 1), (0, 0))))

    rb, svb, st3r = _pass_b(g2, y2, fpre2, wqt, wkbig, wp1big, wp2big, cv, c8,
                            _bd2(p['Wv'].T))
    st3 = st3r[:, :D] + st3r[:, D:]
    a3, b3 = _bn_affine(p['gw0'], p['bw0'], st3[0], st3[1], NK)
    c3 = jnp.stack([_dup(a3), _dup(b3)])

    ww1big = _bd2(p['Ww1'].T)
    st4r = _pass_c(rb, ww1big, c3)
    st4 = st4r[:, :8] + st4r[:, 8:]
    a4, b4 = _bn_affine(p['gw1'], p['bw1'], st4[0], st4[1], NK)

    c16 = jnp.stack([_dup(a4), _dup(b4), _dup(p['bw2']),
                     jnp.zeros(16, jnp.float32)])
    lanes = jnp.arange(2 * D)
    sel16 = (jnp.arange(16)[:, None]
             == jnp.where(lanes < D, lanes % 8, 8 + lanes % 8)[None, :]
             ).astype(jnp.float32)
    xagg2, st5r = _pass_d(rb, svb, ww1big, c3, _bd2(p['Ww2'].T), c16, sel16)
    st5 = st5r[:, :D] + st5r[:, D:]
    a5, b5 = _bn_affine(p['g_bn'], p['b_bn'], st5[0], st5[1], N)

    c5 = jnp.stack([_dup(a5), _dup(b5)])
    x22, st6r = _pass_e2(xagg2, _bd2(p['Wc2'].T), c5)
    st6 = st6r[:, :D] + st6r[:, D:]
    a6, b6 = _bn_affine(p['gc2'], p['bc2'], st6[0], st6[1], N)

    ce = jnp.stack([_dup(a1), _dup(b1), _dup(a6), _dup(b6)])
    out = _pass_e3(fpre2, x22, ce).reshape(N, D)
    return out.T[None, :, :, None]


# speedup vs baseline: 1.4691x; 1.0003x over previous
import functools

import jax
import jax.numpy as jnp
from jax import lax
from jax.experimental import pallas as pl
from jax.experimental.pallas import tpu as pltpu
from jax.experimental.pallas import tpu_sc as plsc

N = 50000
K = 16
D = 64
NK = N * K
EPS = 1e-5

BN2 = 1000
PN = 400
PR2 = PN * K // 2
RB8 = 4000
CHUNK = 128
NW = 32
NCHUNKS = NK // CHUNK


def _lrelu(x):
    return jnp.maximum(x, 0.2 * x)


def _dup(v):
    return jnp.concatenate([v, v])


def _bd2(w):
    a, b = w.shape
    z = jnp.zeros((2 * a, 2 * b), w.dtype)
    return z.at[:a, :b].set(w).at[a:, b:].set(w)


def _sc_row_gather(table, idx, width):
    mesh = plsc.VectorSubcoreMesh(core_axis_name="c", subcore_axis_name="s")
    nj = (NCHUNKS + NW - 1) // NW
    NB = 8

    @functools.partial(
        pl.kernel,
        mesh=mesh,
        compiler_params=pltpu.CompilerParams(use_tc_tiling_on_sc=False),
        out_type=jax.ShapeDtypeStruct((NK, width), jnp.float32),
        scratch_types=(
            [pltpu.VMEM((CHUNK,), jnp.int32)] * NB
            + [pltpu.VMEM((CHUNK, width), jnp.float32)] * NB
            + [pltpu.SemaphoreType.DMA] * NB
        ),
    )
    def k(tab_hbm, idx_hbm, out_hbm, *bufs):
        idxs = bufs[0:NB]
        rows = bufs[NB:2 * NB]
        sems = bufs[2 * NB:3 * NB]
        wid = lax.axis_index("s") * 2 + lax.axis_index("c")

        def issue(j, b):
            c = j * NW + wid

            @pl.when(c < NCHUNKS)
            def _():
                base = c * CHUNK
                pltpu.sync_copy(idx_hbm.at[pl.ds(base, CHUNK)], idxs[b])
                pltpu.async_copy(tab_hbm.at[idxs[b]], rows[b], sems[b])

        def drain(j, b):
            c = j * NW + wid

            @pl.when(c < NCHUNKS)
            def _():
                base = c * CHUNK
                pltpu.make_async_copy(tab_hbm.at[idxs[b]], rows[b],
                                      sems[b]).wait()
                pltpu.sync_copy(rows[b], out_hbm.at[pl.ds(base, CHUNK)])

        for b in range(NB - 1):
            issue(b, b)

        def body(j4, carry):
            j0 = j4 * NB
            for b in range(NB):
                issue(j0 + b + NB - 1, (b + NB - 1) % NB)
                drain(j0 + b, b)
            return carry

        lax.fori_loop(0, (nj + NB - 1) // NB, body, 0)

    return k(table, idx)


def _acc_stats(i, stats_ref, acc, s1, s2):
    @pl.when(i == 0)
    def _():
        acc[...] = jnp.zeros_like(acc)

    acc[0:1, :] += s1
    acc[1:2, :] += s2

    @pl.when(i == pl.num_programs(0) - 1)
    def _():
        stats_ref[...] = acc[...]


def _pass_a(x2, w1big):
    def body(x_ref, w_ref, fpre_ref, stats_ref, acc):
        i = pl.program_id(0)
        t = jnp.dot(x_ref[...], w_ref[...], preferred_element_type=jnp.float32)
        fpre_ref[...] = t
        _acc_stats(i, stats_ref, acc,
                   jnp.sum(t, axis=0, keepdims=True),
                   jnp.sum(t * t, axis=0, keepdims=True))

    return pl.pallas_call(
        body,
        grid=(N // 2 // BN2,),
        in_specs=[
            pl.BlockSpec((BN2, 2 * D), lambda i: (i, 0)),
            pl.BlockSpec((2 * D, 2 * D), lambda i: (0, 0)),
        ],
        out_specs=[
            pl.BlockSpec((BN2, 2 * D), lambda i: (i, 0)),
            pl.BlockSpec((2, 2 * D), lambda i: (0, 0)),
        ],
        out_shape=[
            jax.ShapeDtypeStruct((N // 2, 2 * D), jnp.float32),
            jax.ShapeDtypeStruct((2, 2 * D), jnp.float32),
        ],
        scratch_shapes=[pltpu.VMEM((2, 2 * D), jnp.float32)],
    )(x2, w1big)


def _pass_b0(y8, wp1big8):
    def body(y_ref, w_ref, stats_ref, acc):
        i = pl.program_id(0)
        t = jnp.dot(y_ref[...], w_ref[...], preferred_element_type=jnp.float32)
        _acc_stats(i, stats_ref, acc,
                   jnp.sum(t, axis=0, keepdims=True),
                   jnp.sum(t * t, axis=0, keepdims=True))

    return pl.pallas_call(
        body,
        grid=(NK // 8 // RB8,),
        in_specs=[
            pl.BlockSpec((RB8, 128), lambda i: (i, 0)),
            pl.BlockSpec((128, 32), lambda i: (0, 0)),
        ],
        out_specs=pl.BlockSpec((2, 32), lambda i: (0, 0)),
        out_shape=jax.ShapeDtypeStruct((2, 32), jnp.float32),
        scratch_shapes=[pltpu.VMEM((2, 32), jnp.float32)],
    )(y8, wp1big8)


def _rqk_block(g_ref, y_ref, fp_ref, wq_ref, wk_ref, wp1_ref, wp2_ref,
               cv_ref, c8_ref):
    a1 = cv_ref[0:1, :]
    b1 = cv_ref[1:2, :]
    bq = cv_ref[2:3, :]
    bk = cv_ref[3:4, :]
    bp2 = cv_ref[4:5, :]
    a2 = c8_ref[0:1, :]
    b2 = c8_ref[1:2, :]

    f2 = jnp.maximum(fp_ref[...] * a1 + b1, 0.0)
    q2 = jnp.dot(f2, wq_ref[...], preferred_element_type=jnp.float32) + bq
    qb = jnp.broadcast_to(q2[None], (K, PN // 2, 2 * D)).reshape(PR2, 2 * D)
    fg = jnp.maximum(g_ref[...] * a1 + b1, 0.0)
    fg = fg.reshape(PR2, 2 * D)
    kg = jnp.dot(fg, wk_ref[...], preferred_element_type=jnp.float32) + bk
    y = y_ref[...].reshape(PR2, 32)
    t = jnp.dot(y, wp1_ref[...], preferred_element_type=jnp.float32)
    t = jnp.maximum(t * a2 + b2, 0.0)
    pr = jnp.dot(t, wp2_ref[...], preferred_element_type=jnp.float32) + bp2
    r = kg - qb + pr
    return r, fg, pr


_GATHER_SPECS = [
    pl.BlockSpec((K, PN // 2, 2 * D), lambda i: (0, i, 0)),
    pl.BlockSpec((K, PN // 2, 32), lambda i: (0, i, 0)),
    pl.BlockSpec((PN // 2, 2 * D), lambda i: (i, 0)),
    pl.BlockSpec((2 * D, 2 * D), lambda i: (0, 0)),
    pl.BlockSpec((2 * D, 2 * D), lambda i: (0, 0)),
    pl.BlockSpec((32, 8), lambda i: (0, 0)),
    pl.BlockSpec((8, 2 * D), lambda i: (0, 0)),
    pl.BlockSpec((8, 2 * D), lambda i: (0, 0)),
    pl.BlockSpec((2, 8), lambda i: (0, 0)),
]


def _pass_b(g2, y2, fpre, wqt, wkbig, wp1big, wp2big, cv, c8, wvbig):
    def body(g_ref, y_ref, fpre_ref, wq_ref, wk_ref, wp1_ref, wp2_ref, cv_ref,
             c8_ref, wv_ref, rb_ref, svb_ref, stats_ref, acc):
        i = pl.program_id(0)
        r, fg, pr = _rqk_block(g_ref, y_ref, fpre_ref, wq_ref, wk_ref, wp1_ref,
                               wp2_ref, cv_ref, c8_ref)
        bv = cv_ref[5:6, :]
        vg = jnp.dot(fg, wv_ref[...], preferred_element_type=jnp.float32) + bv
        rb_ref[...] = r.reshape(K, PN // 2, 2 * D).astype(jnp.bfloat16)
        svb_ref[...] = (vg + pr).reshape(K, PN // 2, 2 * D).astype(jnp.bfloat16)
        _acc_stats(i, stats_ref, acc,
                   jnp.sum(r, axis=0, keepdims=True),
                   jnp.sum(r * r, axis=0, keepdims=True))

    return pl.pallas_call(
        body,
        grid=(N // PN,),
        in_specs=_GATHER_SPECS + [
            pl.BlockSpec((2 * D, 2 * D), lambda i: (0, 0)),
        ],
        out_specs=[
            pl.BlockSpec((K, PN // 2, 2 * D), lambda i: (0, i, 0)),
            pl.BlockSpec((K, PN // 2, 2 * D), lambda i: (0, i, 0)),
            pl.BlockSpec((2, 2 * D), lambda i: (0, 0)),
        ],
        out_shape=[
            jax.ShapeDtypeStruct((K, N // 2, 2 * D), jnp.bfloat16),
            jax.ShapeDtypeStruct((K, N // 2, 2 * D), jnp.bfloat16),
            jax.ShapeDtypeStruct((2, 2 * D), jnp.float32),
        ],
        scratch_shapes=[pltpu.VMEM((2, 2 * D), jnp.float32)],
    )(g2, y2, fpre, wqt, wkbig, wp1big, wp2big, cv, c8, wvbig)


def _pass_c(rb, ww1big, c3):
    def body(rb_ref, ww1_ref, c3_ref, stats_ref, acc):
        i = pl.program_id(0)
        r = rb_ref[...].reshape(PR2, 2 * D).astype(jnp.float32)
        u = _lrelu(r * c3_ref[0:1, :] + c3_ref[1:2, :])
        w1 = jnp.dot(u, ww1_ref[...], preferred_element_type=jnp.float32)
        _acc_stats(i, stats_ref, acc,
                   jnp.sum(w1, axis=0, keepdims=True),
                   jnp.sum(w1 * w1, axis=0, keepdims=True))

    return pl.pallas_call(
        body,
        grid=(N // PN,),
        in_specs=[
            pl.BlockSpec((K, PN // 2, 2 * D), lambda i: (0, i, 0)),
            pl.BlockSpec((2 * D, 16), lambda i: (0, 0)),
            pl.BlockSpec((2, 2 * D), lambda i: (0, 0)),
        ],
        out_specs=pl.BlockSpec((2, 16), lambda i: (0, 0)),
        out_shape=jax.ShapeDtypeStruct((2, 16), jnp.float32),
        scratch_shapes=[pltpu.VMEM((2, 16), jnp.float32)],
    )(rb, ww1big, c3)


def _pass_d(rb, svb, ww1big, c3, ww2big, c16, sel16):
    def body(rb_ref, svb_ref, ww1_ref, c3_ref, ww2_ref, c16_ref, sel16_ref,
             x_ref, stats_ref, acc):
        i = pl.program_id(0)
        r = rb_ref[...].reshape(PR2, 2 * D).astype(jnp.float32)
        u = _lrelu(r * c3_ref[0:1, :] + c3_ref[1:2, :])
        w1 = jnp.dot(u, ww1_ref[...], preferred_element_type=jnp.float32)
        u4 = jnp.maximum(w1 * c16_ref[0:1, :] + c16_ref[1:2, :], 0.0)
        w2 = jnp.dot(u4, ww2_ref[...], preferred_element_type=jnp.float32)
        w2 = w2 + c16_ref[2:3, :]
        e = jnp.exp(w2)
        efull = jnp.dot(e, sel16_ref[...],
                        preferred_element_type=jnp.float32,
                        precision=jax.lax.Precision.HIGHEST)
        sv = svb_ref[...].reshape(PR2, 2 * D).astype(jnp.float32)
        p3 = (sv * efull).reshape(K, PN // 2, 2 * D)
        numer = jnp.sum(p3, axis=0)
        zf = jnp.sum(efull.reshape(K, PN // 2, 2 * D), axis=0)
        x = numer / zf
        x_ref[...] = x
        _acc_stats(i, stats_ref, acc,
                   jnp.sum(x, axis=0, keepdims=True),
                   jnp.sum(x * x, axis=0, keepdims=True))

    return pl.pallas_call(
        body,
        grid=(N // PN,),
        in_specs=[
            pl.BlockSpec((K, PN // 2, 2 * D), lambda i: (0, i, 0)),
            pl.BlockSpec((K, PN // 2, 2 * D), lambda i: (0, i, 0)),
            pl.BlockSpec((2 * D, 16), lambda i: (0, 0)),
            pl.BlockSpec((2, 2 * D), lambda i: (0, 0)),
            pl.BlockSpec((16, 16), lambda i: (0, 0)),
            pl.BlockSpec((4, 16), lambda i: (0, 0)),
            pl.BlockSpec((16, 2 * D), lambda i: (0, 0)),
        ],
        out_specs=[
            pl.BlockSpec((PN // 2, 2 * D), lambda i: (i, 0)),
            pl.BlockSpec((2, 2 * D), lambda i: (0, 0)),
        ],
        out_shape=[
            jax.ShapeDtypeStruct((N // 2, 2 * D), jnp.float32),
            jax.ShapeDtypeStruct((2, 2 * D), jnp.float32),
        ],
        scratch_shapes=[pltpu.VMEM((2, 2 * D), jnp.float32)],
    )(rb, svb, ww1big, c3, ww2big, c16, sel16)


def _pass_e2(xagg2, wc2big, c5):
    def body(x_ref, w_ref, c_ref, x2_ref, stats_ref, acc):
        i = pl.program_id(0)
        u = _lrelu(x_ref[...] * c_ref[0:1, :] + c_ref[1:2, :])
        x2 = jnp.dot(u, w_ref[...], preferred_element_type=jnp.float32)
        x2_ref[...] = x2
        _acc_stats(i, stats_ref, acc,
                   jnp.sum(x2, axis=0, keepdims=True),
                   jnp.sum(x2 * x2, axis=0, keepdims=True))

    return pl.pallas_call(
        body,
        grid=(N // 2 // BN2,),
        in_specs=[
            pl.BlockSpec((BN2, 2 * D), lambda i: (i, 0)),
            pl.BlockSpec((2 * D, 2 * D), lambda i: (0, 0)),
            pl.BlockSpec((2, 2 * D), lambda i: (0, 0)),
        ],
        out_specs=[
            pl.BlockSpec((BN2, 2 * D), lambda i: (i, 0)),
            pl.BlockSpec((2, 2 * D), lambda i: (0, 0)),
        ],
        out_shape=[
            jax.ShapeDtypeStruct((N // 2, 2 * D), jnp.float32),
            jax.ShapeDtypeStruct((2, 2 * D), jnp.float32),
        ],
        scratch_shapes=[pltpu.VMEM((2, 2 * D), jnp.float32)],
    )(xagg2, wc2big, c5)


def _pass_e3(fpre2, x22, ce):
    def body(fpre_ref, x2_ref, c_ref, out_ref):
        f = jnp.maximum(fpre_ref[...] * c_ref[0:1, :] + c_ref[1:2, :], 0.0)
        xb = x2_ref[...] * c_ref[2:3, :] + c_ref[3:4, :]
        out_ref[...] = _lrelu(f + xb)

    return pl.pallas_call(
        body,
        grid=(N // 2 // BN2,),
        in_specs=[
            pl.BlockSpec((BN2, 2 * D), lambda i: (i, 0)),
            pl.BlockSpec((BN2, 2 * D), lambda i: (i, 0)),
            pl.BlockSpec((4, 2 * D), lambda i: (0, 0)),
        ],
        out_specs=pl.BlockSpec((BN2, 2 * D), lambda i: (i, 0)),
        out_shape=jax.ShapeDtypeStruct((N // 2, 2 * D), jnp.float32),
    )(fpre2, x22, ce)


def _bn_affine(g, b, s1, s2, m):
    mean = s1 / m
    var = s2 / m - mean * mean
    a = g / jnp.sqrt(var + EPS)
    return a, b - mean * a


def kernel(feature, xyz, params, neigh_idx):
    p = params
    x2 = feature[0, :, :, 0].T.reshape(N // 2, 2 * D)
    xyzp = jnp.pad(xyz[0], ((0, 0), (0, 13)))
    idx = neigh_idx[0].T.reshape(-1).astype(jnp.int32)

    y_flat = _sc_row_gather(xyzp, idx, 16)
    fpre2, st1r = _pass_a(x2, _bd2(p['W1'].T))
    st1 = st1r[:, :D] + st1r[:, D:]
    a1, b1 = _bn_affine(p['g1'], p['b1'], st1[0], st1[1], N)

    g_flat = _sc_row_gather(fpre2.reshape(N, D), idx, D)
    g2 = g_flat.reshape(K, N // 2, 2 * D)
    y2 = y_flat.reshape(K, N // 2, 32)
    y8 = y_flat.reshape(NK // 8, 128)

    wp1t16 = jnp.zeros((16, 4), jnp.float32).at[:3, :3].set(p['Wp1'].T)
    wp1big8 = jax.scipy.linalg.block_diag(*([wp1t16] * 8))
    st2r = _pass_b0(y8, wp1big8)
    st2 = st2r.reshape(2, 8, 4).sum(axis=1)
    g2p = jnp.pad(p['gp1'], (0, 1))
    b2p = jnp.pad(p['bp1'], (0, 1))
    a2, b2 = _bn_affine(g2p, b2p, st2[0], st2[1], NK)

    cv = jnp.stack([_dup(a1), _dup(b1),
                    _dup(p['bq']),
                    _dup(p['bk']), _dup(p['bp2']), _dup(p['bv']),
                    jnp.zeros(2 * D, jnp.float32),
                    jnp.zeros(2 * D, jnp.float32)])
    c8 = jnp.stack([_dup(a2), _dup(b2)])
    wqt = _bd2(p['Wq'].T)
    wkbig = _bd2(p['Wk'].T)
    wp1big = _bd2(wp1t16)
    wp2big = _bd2(jnp.pad(p['Wp2'].T, ((0, 1), (0, 0))))

    rb, svb, st3r = _pass_b(g2, y2, fpre2, wqt, wkbig, wp1big, wp2big, cv, c8,
                            _bd2(p['Wv'].T))
    st3 = st3r[:, :D] + st3r[:, D:]
    a3, b3 = _bn_affine(p['gw0'], p['bw0'], st3[0], st3[1], NK)
    c3 = jnp.stack([_dup(a3), _dup(b3)])

    ww1big = _bd2(p['Ww1'].T)
    st4r = _pass_c(rb, ww1big, c3)
    st4 = st4r[:, :8] + st4r[:, 8:]
    a4, b4 = _bn_affine(p['gw1'], p['bw1'], st4[0], st4[1], NK)

    c16 = jnp.stack([_dup(a4), _dup(b4), _dup(p['bw2']),
                     jnp.zeros(16, jnp.float32)])
    lanes = jnp.arange(2 * D)
    sel16 = (jnp.arange(16)[:, None]
             == jnp.where(lanes < D, lanes % 8, 8 + lanes % 8)[None, :]
             ).astype(jnp.float32)
    xagg2, st5r = _pass_d(rb, svb, ww1big, c3, _bd2(p['Ww2'].T), c16, sel16)
    st5 = st5r[:, :D] + st5r[:, D:]
    a5, b5 = _bn_affine(p['g_bn'], p['b_bn'], st5[0], st5[1], N)

    c5 = jnp.stack([_dup(a5), _dup(b5)])
    x22, st6r = _pass_e2(xagg2, _bd2(p['Wc2'].T), c5)
    st6 = st6r[:, :D] + st6r[:, D:]
    a6, b6 = _bn_affine(p['gc2'], p['bc2'], st6[0], st6[1], N)

    ce = jnp.stack([_dup(a1), _dup(b1), _dup(a6), _dup(b6)])
    out = _pass_e3(fpre2, x22, ce).reshape(N, D)
    return out.T[None, :, :, None]
